# Initial kernel scaffold; baseline (speedup 1.0000x reference)
import jax
import jax.numpy as jnp
from jax.experimental import pallas as pl


def kernel(labels, autoenc_skip0, autoenc_skip1, edges_nn, negs, W0, b0, W1, b1, w_lin):
    raise NotImplementedError("write your pallas kernel here")


# speedup vs baseline: 2.6696x; 2.6696x over previous
import functools

import jax
import jax.numpy as jnp
from jax import lax
from jax.experimental import pallas as pl
from jax.experimental.pallas import tpu as pltpu
from jax.experimental.pallas import tpu_sc as plsc

SIGMA = 0.05
MIX = 0.5
NSP = 2500
B = 4
C = 256
H = 56
OH = 112
NPIX = OH * OH
NNODE = B * NSP
E = 160000
ACCROWS = 5120
MPROWS = 10240
HALF = 128

f32 = jnp.float32
i32 = jnp.int32

_NLP = pltpu.CompilerParams(needs_layout_passes=False)


def _lerp_matrix():
    ys = jnp.linspace(0.0, H - 1.0, OH)
    y0 = jnp.floor(ys).astype(i32)
    y1 = jnp.minimum(y0 + 1, H - 1)
    wy = ys - y0.astype(f32)
    r = jnp.zeros((OH, H), f32)
    rows = jnp.arange(OH)
    r = r.at[rows, y0].add(1.0 - wy)
    r = r.at[rows, y1].add(wy)
    return r


def _aux_rows():
    ii = jnp.repeat(jnp.arange(OH, dtype=f32), OH) / (OH - 1.0)
    jj = jnp.tile(jnp.arange(OH, dtype=f32), OH) / (OH - 1.0)
    a = jnp.zeros((NPIX, HALF), f32)
    return a.at[:, 0].set(1.0).at[:, 1].set(ii).at[:, 2].set(jj)


def _mm_kernel(a_ref, b_ref, o_ref):
    o_ref[...] = jnp.dot(a_ref[...], b_ref[...],
                         preferred_element_type=f32,
                         precision=lax.Precision.HIGHEST)


def _xlerp(x2):
    m = x2.shape[0]
    blk = m // 32
    return pl.pallas_call(
        _mm_kernel,
        grid=(32,),
        in_specs=[pl.BlockSpec((blk, H), lambda i: (i, 0)),
                  pl.BlockSpec((H, OH), lambda i: (0, 0))],
        out_specs=pl.BlockSpec((blk, OH), lambda i: (i, 0)),
        out_shape=jax.ShapeDtypeStruct((m, OH), f32),
    )(x2, _lerp_matrix().T)


def _transpose_kernel(a_ref, e_ref, o_ref):
    o_ref[0] = lax.dot_general(a_ref[0], e_ref[...],
                               (((0,), (0,)), ((), ())),
                               preferred_element_type=f32,
                               precision=lax.Precision.HIGHEST)


def _transpose8(a):
    k = a.shape[2]
    kb = k // 7
    return pl.pallas_call(
        _transpose_kernel,
        grid=(8, 7),
        in_specs=[pl.BlockSpec((1, C, kb), lambda i, j: (i, 0, j)),
                  pl.BlockSpec((C, C), lambda i, j: (0, 0))],
        out_specs=pl.BlockSpec((1, kb, C), lambda i, j: (i, j, 0)),
        out_shape=jax.ShapeDtypeStruct((8, k, C), f32),
    )(a, jnp.eye(C, dtype=f32))


def _ylerp_kernel(r_ref, a_ref, o_ref):
    o_ref[0] = jnp.dot(r_ref[...], a_ref[0],
                       preferred_element_type=f32,
                       precision=lax.Precision.HIGHEST)


def _ylerp(a):
    k = a.shape[2]
    kb = k // 4
    return pl.pallas_call(
        _ylerp_kernel,
        grid=(8, 4),
        in_specs=[pl.BlockSpec((OH, H), lambda i, j: (0, 0)),
                  pl.BlockSpec((1, H, kb), lambda i, j: (i, 0, j))],
        out_specs=pl.BlockSpec((1, OH, kb), lambda i, j: (i, 0, j)),
        out_shape=jax.ShapeDtypeStruct((8, OH, k), f32),
    )(_lerp_matrix(), a)


def _upsample_to_rows(skip0, skip1):
    x = jnp.concatenate([skip0.reshape(B, C, H * H),
                         skip1.reshape(B, C, H * H)], axis=0)
    x2 = x.reshape(8 * C * H, H)
    a = _xlerp(x2)
    a = a.reshape(8, C, H * OH)
    at = _transpose8(a)
    at = at.reshape(8, H, OH * C)
    u = _ylerp(at)
    return u.reshape(8, NPIX, C)


def _xw0_kernel(s_ref, a_ref, w_ref, o_ref, c_ref):
    icnt = 1.0 / jnp.maximum(a_ref[:, 0:1], 1.0)
    xw = (jnp.dot(s_ref[0, 0], w_ref[:HALF], preferred_element_type=f32,
                  precision=lax.Precision.HIGHEST)
          + jnp.dot(s_ref[0, 1], w_ref[HALF:], preferred_element_type=f32,
                    precision=lax.Precision.HIGHEST)) * icnt
    o_ref[0] = xw[:, :HALF]
    o_ref[1] = xw[:, HALF:]
    z = jnp.zeros((a_ref.shape[0], 6), f32)
    c_ref[...] = jnp.concatenate(
        [a_ref[:, 1:2] * icnt, a_ref[:, 2:3] * icnt, z], axis=1)


def _xw1_kernel(g_ref, s_ref, a_ref, b_ref, w_ref, o_ref):
    icnt = 1.0 / jnp.maximum(a_ref[:, 0:1], 1.0)
    g = jnp.concatenate([g_ref[0], g_ref[1]], axis=1)
    s = jnp.concatenate([s_ref[0, 0], s_ref[0, 1]], axis=1)
    x2 = ((1.0 - MIX) * jnp.maximum(g + b_ref[...], 0.0)
          + MIX * icnt * s)
    xw = jnp.dot(x2, w_ref[...], preferred_element_type=f32,
                 precision=lax.Precision.HIGHEST)
    o_ref[0] = xw[:, :HALF]
    o_ref[1] = xw[:, HALF:]


def _y_kernel(g_ref, b_ref, wl_ref, o_ref):
    g = jnp.concatenate([g_ref[0], g_ref[1]], axis=1)
    x3 = jnp.maximum(g + b_ref[...], 0.0)
    o_ref[...] = jnp.sum(x3 * wl_ref[...], axis=1, keepdims=True)


_RB = 2000


def _compute_xw0(ssum, asum, w0):
    return pl.pallas_call(
        _xw0_kernel,
        grid=(NNODE // _RB,),
        in_specs=[pl.BlockSpec((1, 2, _RB, HALF), lambda i: (0, 0, i, 0)),
                  pl.BlockSpec((_RB, HALF), lambda i: (i, 0)),
                  pl.BlockSpec((C, C), lambda i: (0, 0))],
        out_specs=(pl.BlockSpec((2, _RB, HALF), lambda i: (0, i, 0)),
                   pl.BlockSpec((_RB, 8), lambda i: (i, 0))),
        out_shape=(jax.ShapeDtypeStruct((2, NNODE, HALF), f32),
                   jax.ShapeDtypeStruct((NNODE, 8), f32)),
    )(ssum, asum, w0)


def _compute_xw1(agg0, ssum, asum, b0, w1):
    return pl.pallas_call(
        _xw1_kernel,
        grid=(NNODE // _RB,),
        in_specs=[pl.BlockSpec((2, _RB, HALF), lambda i: (0, i, 0)),
                  pl.BlockSpec((1, 2, _RB, HALF), lambda i: (1, 0, i, 0)),
                  pl.BlockSpec((_RB, HALF), lambda i: (i, 0)),
                  pl.BlockSpec((1, C), lambda i: (0, 0)),
                  pl.BlockSpec((C, C), lambda i: (0, 0))],
        out_specs=pl.BlockSpec((2, _RB, HALF), lambda i: (0, i, 0)),
        out_shape=jax.ShapeDtypeStruct((2, NNODE, HALF), f32),
    )(agg0, ssum, asum, b0.reshape(1, C), w1)


def _compute_y(agg1, b1, w_lin):
    return pl.pallas_call(
        _y_kernel,
        grid=(NNODE // _RB,),
        in_specs=[pl.BlockSpec((2, _RB, HALF), lambda i: (0, i, 0)),
                  pl.BlockSpec((1, C), lambda i: (0, 0)),
                  pl.BlockSpec((1, C), lambda i: (0, 0))],
        out_specs=pl.BlockSpec((_RB, 1), lambda i: (i, 0)),
        out_shape=jax.ShapeDtypeStruct((NNODE, 1), f32),
    )(agg1, b1.reshape(1, C), w_lin.reshape(1, C))


@functools.cache
def _sc_mesh():
    return plsc.VectorSubcoreMesh(core_axis_name="c", subcore_axis_name="s")


_PCH = 112
_PPT = NPIX // 16
_ZR = ACCROWS // 16


def _pool_body(u_hbm, lab_hbm, aux_hbm, zacc_hbm, ssum_hbm, asum_hbm,
               acc, labv, rows, sem):
    cid = lax.axis_index("c")
    sid = lax.axis_index("s")

    def scatter_img(img, src_fn):
        for k in range(_PPT // _PCH):
            base = sid * _PPT + k * _PCH
            pltpu.sync_copy(lab_hbm.at[pl.ds(img * NPIX + base, _PCH)],
                            labv)
            src_fn(base)
            pltpu.async_copy(rows, acc.at[labv], sem, add=True).wait()

    def drain(out_at):
        pltpu.sync_copy(acc.at[pl.ds(sid * 312, 312)],
                        out_at(sid * 312, 312))

        @pl.when(sid == 0)
        def _():
            pltpu.sync_copy(acc.at[pl.ds(4992, 8)], out_at(4992, 8))

    for m in range(2):
        for h in range(2):
            pltpu.sync_copy(zacc_hbm.at[pl.ds(sid * _ZR, _ZR)],
                            acc.at[pl.ds(sid * _ZR, _ZR)])
            plsc.subcore_barrier()
            for bl in range(2):
                img = cid * 2 + bl
                mi = m * 4 + img

                def src(base, mi=mi):
                    pltpu.sync_copy(u_hbm.at[mi, pl.ds(base, _PCH), h], rows)

                scatter_img(img, src)
            plsc.subcore_barrier()
            drain(lambda r, n: ssum_hbm.at[m, h, pl.ds(cid * 5000 + r, n)])
            plsc.subcore_barrier()

    pltpu.sync_copy(zacc_hbm.at[pl.ds(sid * _ZR, _ZR)],
                    acc.at[pl.ds(sid * _ZR, _ZR)])
    plsc.subcore_barrier()
    for bl in range(2):
        img = cid * 2 + bl

        def asrc(base):
            pltpu.sync_copy(aux_hbm.at[pl.ds(base, _PCH)], rows)

        scatter_img(img, asrc)
    plsc.subcore_barrier()
    drain(lambda r, n: asum_hbm.at[pl.ds(cid * 5000 + r, n)])


def _sc_pool(u4, labels2, aux, zacc):
    fn = pl.kernel(
        _pool_body,
        out_type=(jax.ShapeDtypeStruct((2, 2, NNODE, HALF), f32),
                  jax.ShapeDtypeStruct((NNODE, HALF), f32)),
        mesh=_sc_mesh(),
        scratch_types=[
            pltpu.VMEM_SHARED((ACCROWS, HALF), f32),
            pltpu.VMEM((_PCH,), i32),
            pltpu.VMEM((_PCH, HALF), f32),
            pltpu.SemaphoreType.DMA,
        ],
        compiler_params=_NLP,
    )
    return fn(u4, labels2, aux, zacc)


_EPW = E // 32
_EVR = _EPW // 16


def _sanitize_tail(ref):
    lanes = lax.iota(i32, 16)
    v = ref[pl.ds(4992, 16)]
    ref[pl.ds(4992, 16)] = jnp.where(lanes < 8, v, 0)


def _eattr_body(e0_hbm, e1_hbm, tab_hbm, ea_hbm, tab, e0v, e1v, outv):
    cid = lax.axis_index("c")
    sid = lax.axis_index("s")
    wid = sid * 2 + cid
    base = wid * _EPW
    pltpu.sync_copy(tab_hbm, tab)
    pltpu.sync_copy(e0_hbm.at[pl.ds(base, _EPW)], e0v.at[pl.ds(0, _EPW)])
    pltpu.sync_copy(e1_hbm.at[pl.ds(base, _EPW)], e1v.at[pl.ds(0, _EPW)])
    _sanitize_tail(e0v)
    _sanitize_tail(e1v)

    def body(i, carry):
        sl = pl.ds(i * 16, 16)
        a = e0v[sl] * 8
        b = e1v[sl] * 8
        dx = plsc.load_gather(tab, [a]) - plsc.load_gather(tab, [b])
        dy = plsc.load_gather(tab, [a + 1]) - plsc.load_gather(tab, [b + 1])
        outv[sl] = jnp.exp(-(dx * dx + dy * dy) * (1.0 / SIGMA))
        return carry

    lax.fori_loop(0, _EVR + 1, body, 0)
    pltpu.sync_copy(outv.at[pl.ds(0, _EPW)], ea_hbm.at[pl.ds(base, _EPW)])


def _sc_edge_attr(e0, e1, tabflat):
    fn = pl.kernel(
        _eattr_body,
        out_type=jax.ShapeDtypeStruct((E,), f32),
        mesh=_sc_mesh(),
        scratch_types=[
            pltpu.VMEM((NNODE * 8,), f32),
            pltpu.VMEM((_EPW + 16,), i32),
            pltpu.VMEM((_EPW + 16,), i32),
            pltpu.VMEM((_EPW + 16,), f32),
        ],
        compiler_params=_NLP,
    )
    return fn(e0, e1, tabflat)


_ECH = 80
_EPT = E // 16


def _mp_body(xw_hbm, e0_hbm, e1_hbm, ea_hbm, zmp_hbm, agg_hbm,
             acc, srcv, dstv, eav, rows, sem):
    cid = lax.axis_index("c")
    sid = lax.axis_index("s")
    pltpu.sync_copy(zmp_hbm.at[pl.ds(sid * 640, 640)],
                    acc.at[pl.ds(sid * 640, 640)])
    plsc.subcore_barrier()

    def chunk(k, carry):
        base = sid * _EPT + k * _ECH
        pltpu.sync_copy(e0_hbm.at[pl.ds(base, _ECH)], srcv)
        pltpu.sync_copy(e1_hbm.at[pl.ds(base, _ECH)], dstv)
        pltpu.sync_copy(ea_hbm.at[pl.ds(base, _ECH)], eav)
        for t in range(_ECH // 16):
            srcv[pl.ds(t * 16, 16)] = (srcv[pl.ds(t * 16, 16)]
                                       + cid * NNODE)
        pltpu.async_copy(xw_hbm.at[srcv], rows, sem).wait()

        def edge(j, c2):
            s = plsc.load_gather(eav, [jnp.full((16,), 0, i32) + j])
            for q in range(HALF // 16):
                sl = pl.ds(q * 16, 16)
                rows[j, sl] = rows[j, sl] * s
            return c2

        lax.fori_loop(0, _ECH, edge, 0)
        pltpu.async_copy(rows, acc.at[dstv], sem, add=True).wait()
        return carry

    lax.fori_loop(0, _EPT // _ECH, chunk, 0)
    plsc.subcore_barrier()
    pltpu.sync_copy(acc.at[pl.ds(sid * 624, 624)],
                    agg_hbm.at[cid, pl.ds(sid * 624, 624)])

    @pl.when(sid == 0)
    def _():
        pltpu.sync_copy(acc.at[pl.ds(9984, 16)],
                        agg_hbm.at[cid, pl.ds(9984, 16)])


def _sc_message_pass(xwflat, e0, e1, ea, zmp):
    fn = pl.kernel(
        _mp_body,
        out_type=jax.ShapeDtypeStruct((2, NNODE, HALF), f32),
        mesh=_sc_mesh(),
        scratch_types=[
            pltpu.VMEM_SHARED((MPROWS, HALF), f32),
            pltpu.VMEM((_ECH,), i32),
            pltpu.VMEM((_ECH,), i32),
            pltpu.VMEM((_ECH,), f32),
            pltpu.VMEM((_ECH, HALF), f32),
            pltpu.SemaphoreType.DMA,
        ],
        compiler_params=_NLP,
    )
    return fn(xwflat, e0, e1, ea, zmp)


def _scores_body(y_hbm, e0_hbm, e1_hbm, negs_hbm, dan_hbm, dap_hbm,
                 yv, e0v, e1v, env, danv, dapv):
    cid = lax.axis_index("c")
    sid = lax.axis_index("s")
    wid = sid * 2 + cid
    base = wid * _EPW
    pltpu.sync_copy(y_hbm, yv)
    pltpu.sync_copy(e0_hbm.at[pl.ds(base, _EPW)], e0v.at[pl.ds(0, _EPW)])
    pltpu.sync_copy(e1_hbm.at[pl.ds(base, _EPW)], e1v.at[pl.ds(0, _EPW)])
    pltpu.sync_copy(negs_hbm.at[pl.ds(base, _EPW)], env.at[pl.ds(0, _EPW)])
    _sanitize_tail(e0v)
    _sanitize_tail(e1v)
    _sanitize_tail(env)

    def body(i, carry):
        sl = pl.ds(i * 16, 16)
        s0 = plsc.load_gather(yv, [e0v[sl]])
        s1 = plsc.load_gather(yv, [e1v[sl]])
        sn = plsc.load_gather(yv, [env[sl]])
        dapv[sl] = 1.0 / (1.0 + jnp.exp(s1 - s0))
        danv[sl] = 1.0 / (1.0 + jnp.exp(sn - s0))
        return carry

    lax.fori_loop(0, _EVR + 1, body, 0)
    pltpu.sync_copy(danv.at[pl.ds(0, _EPW)], dan_hbm.at[pl.ds(base, _EPW)])
    pltpu.sync_copy(dapv.at[pl.ds(0, _EPW)], dap_hbm.at[pl.ds(base, _EPW)])


def _sc_scores(y, e0, e1, negs):
    fn = pl.kernel(
        _scores_body,
        out_type=(jax.ShapeDtypeStruct((E,), f32),
                  jax.ShapeDtypeStruct((E,), f32)),
        mesh=_sc_mesh(),
        scratch_types=[
            pltpu.VMEM((NNODE,), f32),
            pltpu.VMEM((_EPW + 16,), i32),
            pltpu.VMEM((_EPW + 16,), i32),
            pltpu.VMEM((_EPW + 16,), i32),
            pltpu.VMEM((_EPW + 16,), f32),
            pltpu.VMEM((_EPW + 16,), f32),
        ],
        compiler_params=_NLP,
    )
    return fn(y, e0, e1, negs)


def kernel(labels, autoenc_skip0, autoenc_skip1, edges_nn, negs,
           W0, b0, W1, b1, w_lin):
    off = jnp.array([0, NSP, 0, NSP], i32).reshape(B, 1)
    labels2 = (labels.reshape(B, NPIX).astype(i32) + off).reshape(B * NPIX)
    e0 = edges_nn[0].astype(i32)
    e1 = edges_nn[1].astype(i32)
    negs = negs.astype(i32)

    u = _upsample_to_rows(autoenc_skip0, autoenc_skip1)
    u4 = u.reshape(8, NPIX, 2, HALF)

    zacc = jnp.zeros((ACCROWS, HALF), f32)
    zmp = jnp.zeros((MPROWS, HALF), f32)

    ssum, asum = _sc_pool(u4, labels2, _aux_rows(), zacc)
    xw0, coords = _compute_xw0(ssum, asum, W0)
    ea = _sc_edge_attr(e0, e1, coords.reshape(NNODE * 8))

    agg0 = _sc_message_pass(xw0.reshape(2 * NNODE, HALF), e0, e1, ea, zmp)
    xw1 = _compute_xw1(agg0, ssum, asum, b0, W1)
    agg1 = _sc_message_pass(xw1.reshape(2 * NNODE, HALF), e0, e1, ea, zmp)
    y = _compute_y(agg1, b1, w_lin)

    dan, dap = _sc_scores(y.reshape(NNODE), e0, e1, negs)
    return (dan.reshape(E, 1), dap.reshape(E, 1), ea)


# speedup vs baseline: 3.1981x; 1.1980x over previous
import functools

import jax
import jax.numpy as jnp
from jax import lax
from jax.experimental import pallas as pl
from jax.experimental.pallas import tpu as pltpu
from jax.experimental.pallas import tpu_sc as plsc

SIGMA = 0.05
MIX = 0.5
NSP = 2500
B = 4
C = 256
H = 56
OH = 112
NPIX = OH * OH
NNODE = B * NSP
E = 160000
ACCROWS = 5120
MPROWS = 10240
HALF = 128

f32 = jnp.float32
i32 = jnp.int32

_NLP = pltpu.CompilerParams(needs_layout_passes=False)


def _lerp_matrix():
    ys = jnp.linspace(0.0, H - 1.0, OH)
    y0 = jnp.floor(ys).astype(i32)
    y1 = jnp.minimum(y0 + 1, H - 1)
    wy = ys - y0.astype(f32)
    r = jnp.zeros((OH, H), f32)
    rows = jnp.arange(OH)
    r = r.at[rows, y0].add(1.0 - wy)
    r = r.at[rows, y1].add(wy)
    return r


def _aux_rows():
    ii = jnp.repeat(jnp.arange(OH, dtype=f32), OH) / (OH - 1.0)
    jj = jnp.tile(jnp.arange(OH, dtype=f32), OH) / (OH - 1.0)
    a = jnp.zeros((NPIX, HALF), f32)
    return a.at[:, 0].set(1.0).at[:, 1].set(ii).at[:, 2].set(jj)


def _mm_kernel(a_ref, b_ref, o_ref):
    o_ref[...] = jnp.dot(a_ref[...], b_ref[...],
                         preferred_element_type=f32,
                         precision=lax.Precision.HIGHEST)


def _xlerp(x2):
    m = x2.shape[0]
    blk = m // 32
    return pl.pallas_call(
        _mm_kernel,
        grid=(32,),
        in_specs=[pl.BlockSpec((blk, H), lambda i: (i, 0)),
                  pl.BlockSpec((H, OH), lambda i: (0, 0))],
        out_specs=pl.BlockSpec((blk, OH), lambda i: (i, 0)),
        out_shape=jax.ShapeDtypeStruct((m, OH), f32),
    )(x2, _lerp_matrix().T)


def _transpose_kernel(a_ref, e_ref, o_ref):
    o_ref[0] = lax.dot_general(a_ref[0], e_ref[...],
                               (((0,), (0,)), ((), ())),
                               preferred_element_type=f32,
                               precision=lax.Precision.HIGHEST)


def _transpose8(a):
    k = a.shape[2]
    kb = k // 7
    return pl.pallas_call(
        _transpose_kernel,
        grid=(8, 7),
        in_specs=[pl.BlockSpec((1, C, kb), lambda i, j: (i, 0, j)),
                  pl.BlockSpec((C, C), lambda i, j: (0, 0))],
        out_specs=pl.BlockSpec((1, kb, C), lambda i, j: (i, j, 0)),
        out_shape=jax.ShapeDtypeStruct((8, k, C), f32),
    )(a, jnp.eye(C, dtype=f32))


def _ylerp_kernel(r_ref, a_ref, o_ref):
    o_ref[0] = jnp.dot(r_ref[...], a_ref[0],
                       preferred_element_type=f32,
                       precision=lax.Precision.HIGHEST)


def _ylerp(a):
    k = a.shape[2]
    kb = k // 4
    return pl.pallas_call(
        _ylerp_kernel,
        grid=(8, 4),
        in_specs=[pl.BlockSpec((OH, H), lambda i, j: (0, 0)),
                  pl.BlockSpec((1, H, kb), lambda i, j: (i, 0, j))],
        out_specs=pl.BlockSpec((1, OH, kb), lambda i, j: (i, 0, j)),
        out_shape=jax.ShapeDtypeStruct((8, OH, k), f32),
    )(_lerp_matrix(), a)


def _upsample_to_rows(skip0, skip1):
    x = jnp.concatenate([skip0.reshape(B, C, H * H),
                         skip1.reshape(B, C, H * H)], axis=0)
    x2 = x.reshape(8 * C * H, H)
    a = _xlerp(x2)
    a = a.reshape(8, C, H * OH)
    at = _transpose8(a)
    at = at.reshape(8, H, OH * C)
    u = _ylerp(at)
    return u.reshape(8, NPIX, C)


def _xw0_kernel(s_ref, a_ref, w_ref, o_ref, c_ref):
    icnt = 1.0 / jnp.maximum(a_ref[:, 0:1], 1.0)
    xw = (jnp.dot(s_ref[0, 0], w_ref[:HALF], preferred_element_type=f32,
                  precision=lax.Precision.HIGHEST)
          + jnp.dot(s_ref[0, 1], w_ref[HALF:], preferred_element_type=f32,
                    precision=lax.Precision.HIGHEST)) * icnt
    o_ref[0] = xw[:, :HALF]
    o_ref[1] = xw[:, HALF:]
    z = jnp.zeros((a_ref.shape[0], 6), f32)
    c_ref[...] = jnp.concatenate(
        [a_ref[:, 1:2] * icnt, a_ref[:, 2:3] * icnt, z], axis=1)


def _xw1_kernel(g_ref, s_ref, a_ref, b_ref, w_ref, o_ref):
    icnt = 1.0 / jnp.maximum(a_ref[:, 0:1], 1.0)
    g = jnp.concatenate([g_ref[0], g_ref[1]], axis=1)
    s = jnp.concatenate([s_ref[0, 0], s_ref[0, 1]], axis=1)
    x2 = ((1.0 - MIX) * jnp.maximum(g + b_ref[...], 0.0)
          + MIX * icnt * s)
    xw = jnp.dot(x2, w_ref[...], preferred_element_type=f32,
                 precision=lax.Precision.HIGHEST)
    o_ref[0] = xw[:, :HALF]
    o_ref[1] = xw[:, HALF:]


def _y_kernel(g_ref, b_ref, wl_ref, o_ref):
    g = jnp.concatenate([g_ref[0], g_ref[1]], axis=1)
    x3 = jnp.maximum(g + b_ref[...], 0.0)
    o_ref[...] = jnp.sum(x3 * wl_ref[...], axis=1, keepdims=True)


_RB = 2000


def _compute_xw0(ssum, asum, w0):
    return pl.pallas_call(
        _xw0_kernel,
        grid=(NNODE // _RB,),
        in_specs=[pl.BlockSpec((1, 2, _RB, HALF), lambda i: (0, 0, i, 0)),
                  pl.BlockSpec((_RB, HALF), lambda i: (i, 0)),
                  pl.BlockSpec((C, C), lambda i: (0, 0))],
        out_specs=(pl.BlockSpec((2, _RB, HALF), lambda i: (0, i, 0)),
                   pl.BlockSpec((_RB, 8), lambda i: (i, 0))),
        out_shape=(jax.ShapeDtypeStruct((2, NNODE, HALF), f32),
                   jax.ShapeDtypeStruct((NNODE, 8), f32)),
    )(ssum, asum, w0)


def _compute_xw1(agg0, ssum, asum, b0, w1):
    return pl.pallas_call(
        _xw1_kernel,
        grid=(NNODE // _RB,),
        in_specs=[pl.BlockSpec((2, _RB, HALF), lambda i: (0, i, 0)),
                  pl.BlockSpec((1, 2, _RB, HALF), lambda i: (1, 0, i, 0)),
                  pl.BlockSpec((_RB, HALF), lambda i: (i, 0)),
                  pl.BlockSpec((1, C), lambda i: (0, 0)),
                  pl.BlockSpec((C, C), lambda i: (0, 0))],
        out_specs=pl.BlockSpec((2, _RB, HALF), lambda i: (0, i, 0)),
        out_shape=jax.ShapeDtypeStruct((2, NNODE, HALF), f32),
    )(agg0, ssum, asum, b0.reshape(1, C), w1)


def _compute_y(agg1, b1, w_lin):
    return pl.pallas_call(
        _y_kernel,
        grid=(NNODE // _RB,),
        in_specs=[pl.BlockSpec((2, _RB, HALF), lambda i: (0, i, 0)),
                  pl.BlockSpec((1, C), lambda i: (0, 0)),
                  pl.BlockSpec((1, C), lambda i: (0, 0))],
        out_specs=pl.BlockSpec((_RB, 1), lambda i: (i, 0)),
        out_shape=jax.ShapeDtypeStruct((NNODE, 1), f32),
    )(agg1, b1.reshape(1, C), w_lin.reshape(1, C))


@functools.cache
def _sc_mesh():
    return plsc.VectorSubcoreMesh(core_axis_name="c", subcore_axis_name="s")


_PCH = 112
_PPT = NPIX // 16
_ZR = ACCROWS // 16


def _pool_body(u_hbm, lab_hbm, aux_hbm, zacc_hbm, ssum_hbm, asum_hbm,
               acc, lab0, row0, gs0, ss0, lab1, row1, gs1, ss1):
    cid = lax.axis_index("c")
    sid = lax.axis_index("s")
    bufs = ((lab0, row0, gs0, ss0), (lab1, row1, gs1, ss1))
    gdesc = [None, None]
    sdesc = [None, None]

    def fetch(chunk, b):
        img, src_at, base = chunk
        labv, rows, gs, _ = bufs[b]
        if sdesc[b] is not None:
            sdesc[b].wait()
            sdesc[b] = None
        pltpu.sync_copy(lab_hbm.at[pl.ds(img * NPIX + base, _PCH)], labv)
        gdesc[b] = pltpu.async_copy(src_at(base), rows, gs)

    def scatter(b):
        labv, rows, _, ss = bufs[b]
        gdesc[b].wait()
        gdesc[b] = None
        sdesc[b] = pltpu.async_copy(rows, acc.at[labv], ss, add=True)

    def run_phase(chunks):
        fetch(chunks[0], 0)
        for i in range(len(chunks)):
            if i + 1 < len(chunks):
                fetch(chunks[i + 1], (i + 1) % 2)
            scatter(i % 2)
        for b in range(2):
            if sdesc[b] is not None:
                sdesc[b].wait()
                sdesc[b] = None

    def drain(out_at):
        pltpu.sync_copy(acc.at[pl.ds(sid * 312, 312)],
                        out_at(sid * 312, 312))

        @pl.when(sid == 0)
        def _():
            pltpu.sync_copy(acc.at[pl.ds(4992, 8)], out_at(4992, 8))

    def phase_chunks(src_for_img):
        chunks = []
        for bl in range(2):
            img = cid * 2 + bl
            src_at = src_for_img(bl)
            for k in range(_PPT // _PCH):
                chunks.append((img, src_at, sid * _PPT + k * _PCH))
        return chunks

    for m in range(2):
        for h in range(2):
            pltpu.sync_copy(zacc_hbm.at[pl.ds(sid * _ZR, _ZR)],
                            acc.at[pl.ds(sid * _ZR, _ZR)])
            plsc.subcore_barrier()

            def usrc(bl, m=m, h=h):
                mi = m * 4 + cid * 2 + bl
                return lambda base: u_hbm.at[mi, pl.ds(base, _PCH), h]

            run_phase(phase_chunks(usrc))
            plsc.subcore_barrier()
            drain(lambda r, n: ssum_hbm.at[m, h, pl.ds(cid * 5000 + r, n)])
            plsc.subcore_barrier()

    pltpu.sync_copy(zacc_hbm.at[pl.ds(sid * _ZR, _ZR)],
                    acc.at[pl.ds(sid * _ZR, _ZR)])
    plsc.subcore_barrier()
    run_phase(phase_chunks(
        lambda bl: lambda base: aux_hbm.at[pl.ds(base, _PCH)]))
    plsc.subcore_barrier()
    drain(lambda r, n: asum_hbm.at[pl.ds(cid * 5000 + r, n)])


def _sc_pool(u4, labels2, aux, zacc):
    fn = pl.kernel(
        _pool_body,
        out_type=(jax.ShapeDtypeStruct((2, 2, NNODE, HALF), f32),
                  jax.ShapeDtypeStruct((NNODE, HALF), f32)),
        mesh=_sc_mesh(),
        scratch_types=[
            pltpu.VMEM_SHARED((ACCROWS, HALF), f32),
            pltpu.VMEM((_PCH,), i32),
            pltpu.VMEM((_PCH, HALF), f32),
            pltpu.SemaphoreType.DMA,
            pltpu.SemaphoreType.DMA,
            pltpu.VMEM((_PCH,), i32),
            pltpu.VMEM((_PCH, HALF), f32),
            pltpu.SemaphoreType.DMA,
            pltpu.SemaphoreType.DMA,
        ],
        compiler_params=_NLP,
    )
    return fn(u4, labels2, aux, zacc)


_EPW = E // 32
_EVR = _EPW // 16


def _sanitize_tail(ref):
    lanes = lax.iota(i32, 16)
    v = ref[pl.ds(4992, 16)]
    ref[pl.ds(4992, 16)] = jnp.where(lanes < 8, v, 0)


def _eattr_body(e0_hbm, e1_hbm, tab_hbm, ea_hbm, tab, e0v, e1v, outv):
    cid = lax.axis_index("c")
    sid = lax.axis_index("s")
    wid = sid * 2 + cid
    base = wid * _EPW
    pltpu.sync_copy(tab_hbm, tab)
    pltpu.sync_copy(e0_hbm.at[pl.ds(base, _EPW)], e0v.at[pl.ds(0, _EPW)])
    pltpu.sync_copy(e1_hbm.at[pl.ds(base, _EPW)], e1v.at[pl.ds(0, _EPW)])
    _sanitize_tail(e0v)
    _sanitize_tail(e1v)

    def body(i, carry):
        sl = pl.ds(i * 16, 16)
        a = e0v[sl] * 8
        b = e1v[sl] * 8
        dx = plsc.load_gather(tab, [a]) - plsc.load_gather(tab, [b])
        dy = plsc.load_gather(tab, [a + 1]) - plsc.load_gather(tab, [b + 1])
        outv[sl] = jnp.exp(-(dx * dx + dy * dy) * (1.0 / SIGMA))
        return carry

    lax.fori_loop(0, _EVR + 1, body, 0)
    pltpu.sync_copy(outv.at[pl.ds(0, _EPW)], ea_hbm.at[pl.ds(base, _EPW)])


def _sc_edge_attr(e0, e1, tabflat):
    fn = pl.kernel(
        _eattr_body,
        out_type=jax.ShapeDtypeStruct((E,), f32),
        mesh=_sc_mesh(),
        scratch_types=[
            pltpu.VMEM((NNODE * 8,), f32),
            pltpu.VMEM((_EPW + 16,), i32),
            pltpu.VMEM((_EPW + 16,), i32),
            pltpu.VMEM((_EPW + 16,), f32),
        ],
        compiler_params=_NLP,
    )
    return fn(e0, e1, tabflat)


_ECH = 80
_EPT = E // 16
_NCH = _EPT // _ECH


def _mp_body(xw_hbm, e0_hbm, e1_hbm, ea_hbm, zmp_hbm, agg_hbm, acc,
             s0, d0, w0v, r0, gs0, ss0, s1, d1, w1v, r1, gs1, ss1):
    cid = lax.axis_index("c")
    sid = lax.axis_index("s")
    pltpu.sync_copy(zmp_hbm.at[pl.ds(sid * 640, 640)],
                    acc.at[pl.ds(sid * 640, 640)])
    plsc.subcore_barrier()
    bufs = ((s0, d0, w0v, r0, gs0, ss0), (s1, d1, w1v, r1, gs1, ss1))

    def fetch(k, b):
        sv, dv, wv, rw, gs, _ = bufs[b]
        base = sid * _EPT + k * _ECH
        pltpu.sync_copy(e0_hbm.at[pl.ds(base, _ECH)], sv)
        pltpu.sync_copy(e1_hbm.at[pl.ds(base, _ECH)], dv)
        pltpu.sync_copy(ea_hbm.at[pl.ds(base, _ECH)], wv)
        for t in range(_ECH // 16):
            sv[pl.ds(t * 16, 16)] = sv[pl.ds(t * 16, 16)] + cid * NNODE
        pltpu.async_copy(xw_hbm.at[sv], rw, gs)

    def compute(b):
        sv, dv, wv, rw, gs, ss = bufs[b]
        pltpu.make_async_copy(xw_hbm.at[sv], rw, gs).wait()

        def edge(j, c2):
            s = plsc.load_gather(wv, [jnp.full((16,), 0, i32) + j])
            for q in range(HALF // 16):
                sl = pl.ds(q * 16, 16)
                rw[j, sl] = rw[j, sl] * s
            return c2

        lax.fori_loop(0, _ECH, edge, 0)
        pltpu.async_copy(rw, acc.at[dv], ss, add=True)

    def wait_scatter(b):
        _, dv, _, rw, _, ss = bufs[b]
        pltpu.make_async_copy(rw, acc.at[dv], ss).wait()

    fetch(0, 0)

    def k2body(k2, carry):
        @pl.when(k2 > 0)
        def _():
            wait_scatter(1)

        fetch(2 * k2 + 1, 1)
        compute(0)
        wait_scatter(0)
        fetch(2 * k2 + 2, 0)
        compute(1)
        return carry

    lax.fori_loop(0, (_NCH - 1) // 2, k2body, 0)
    wait_scatter(1)
    compute(0)
    wait_scatter(0)
    plsc.subcore_barrier()
    pltpu.sync_copy(acc.at[pl.ds(sid * 624, 624)],
                    agg_hbm.at[cid, pl.ds(sid * 624, 624)])

    @pl.when(sid == 0)
    def _():
        pltpu.sync_copy(acc.at[pl.ds(9984, 16)],
                        agg_hbm.at[cid, pl.ds(9984, 16)])


def _sc_message_pass(xwflat, e0, e1, ea, zmp):
    fn = pl.kernel(
        _mp_body,
        out_type=jax.ShapeDtypeStruct((2, NNODE, HALF), f32),
        mesh=_sc_mesh(),
        scratch_types=[
            pltpu.VMEM_SHARED((MPROWS, HALF), f32),
            pltpu.VMEM((_ECH,), i32),
            pltpu.VMEM((_ECH,), i32),
            pltpu.VMEM((_ECH,), f32),
            pltpu.VMEM((_ECH, HALF), f32),
            pltpu.SemaphoreType.DMA,
            pltpu.SemaphoreType.DMA,
            pltpu.VMEM((_ECH,), i32),
            pltpu.VMEM((_ECH,), i32),
            pltpu.VMEM((_ECH,), f32),
            pltpu.VMEM((_ECH, HALF), f32),
            pltpu.SemaphoreType.DMA,
            pltpu.SemaphoreType.DMA,
        ],
        compiler_params=_NLP,
    )
    return fn(xwflat, e0, e1, ea, zmp)


def _scores_body(y_hbm, e0_hbm, e1_hbm, negs_hbm, dan_hbm, dap_hbm,
                 yv, e0v, e1v, env, danv, dapv):
    cid = lax.axis_index("c")
    sid = lax.axis_index("s")
    wid = sid * 2 + cid
    base = wid * _EPW
    pltpu.sync_copy(y_hbm, yv)
    pltpu.sync_copy(e0_hbm.at[pl.ds(base, _EPW)], e0v.at[pl.ds(0, _EPW)])
    pltpu.sync_copy(e1_hbm.at[pl.ds(base, _EPW)], e1v.at[pl.ds(0, _EPW)])
    pltpu.sync_copy(negs_hbm.at[pl.ds(base, _EPW)], env.at[pl.ds(0, _EPW)])
    _sanitize_tail(e0v)
    _sanitize_tail(e1v)
    _sanitize_tail(env)

    def body(i, carry):
        sl = pl.ds(i * 16, 16)
        s0 = plsc.load_gather(yv, [e0v[sl]])
        s1 = plsc.load_gather(yv, [e1v[sl]])
        sn = plsc.load_gather(yv, [env[sl]])
        dapv[sl] = 1.0 / (1.0 + jnp.exp(s1 - s0))
        danv[sl] = 1.0 / (1.0 + jnp.exp(sn - s0))
        return carry

    lax.fori_loop(0, _EVR + 1, body, 0)
    pltpu.sync_copy(danv.at[pl.ds(0, _EPW)], dan_hbm.at[pl.ds(base, _EPW)])
    pltpu.sync_copy(dapv.at[pl.ds(0, _EPW)], dap_hbm.at[pl.ds(base, _EPW)])


def _sc_scores(y, e0, e1, negs):
    fn = pl.kernel(
        _scores_body,
        out_type=(jax.ShapeDtypeStruct((E,), f32),
                  jax.ShapeDtypeStruct((E,), f32)),
        mesh=_sc_mesh(),
        scratch_types=[
            pltpu.VMEM((NNODE,), f32),
            pltpu.VMEM((_EPW + 16,), i32),
            pltpu.VMEM((_EPW + 16,), i32),
            pltpu.VMEM((_EPW + 16,), i32),
            pltpu.VMEM((_EPW + 16,), f32),
            pltpu.VMEM((_EPW + 16,), f32),
        ],
        compiler_params=_NLP,
    )
    return fn(y, e0, e1, negs)


def kernel(labels, autoenc_skip0, autoenc_skip1, edges_nn, negs,
           W0, b0, W1, b1, w_lin):
    off = jnp.array([0, NSP, 0, NSP], i32).reshape(B, 1)
    labels2 = (labels.reshape(B, NPIX).astype(i32) + off).reshape(B * NPIX)
    e0 = edges_nn[0].astype(i32)
    e1 = edges_nn[1].astype(i32)
    negs = negs.astype(i32)

    u = _upsample_to_rows(autoenc_skip0, autoenc_skip1)
    u4 = u.reshape(8, NPIX, 2, HALF)

    zacc = jnp.zeros((ACCROWS, HALF), f32)
    zmp = jnp.zeros((MPROWS, HALF), f32)

    ssum, asum = _sc_pool(u4, labels2, _aux_rows(), zacc)
    xw0, coords = _compute_xw0(ssum, asum, W0)
    ea = _sc_edge_attr(e0, e1, coords.reshape(NNODE * 8))

    agg0 = _sc_message_pass(xw0.reshape(2 * NNODE, HALF), e0, e1, ea, zmp)
    xw1 = _compute_xw1(agg0, ssum, asum, b0, W1)
    agg1 = _sc_message_pass(xw1.reshape(2 * NNODE, HALF), e0, e1, ea, zmp)
    y = _compute_y(agg1, b1, w_lin)

    dan, dap = _sc_scores(y.reshape(NNODE), e0, e1, negs)
    return (dan.reshape(E, 1), dap.reshape(E, 1), ea)


# speedup vs baseline: 3.2005x; 1.0008x over previous
import functools

import jax
import jax.numpy as jnp
from jax import lax
from jax.experimental import pallas as pl
from jax.experimental.pallas import tpu as pltpu
from jax.experimental.pallas import tpu_sc as plsc

SIGMA = 0.05
MIX = 0.5
NSP = 2500
B = 4
C = 256
H = 56
OH = 112
NPIX = OH * OH
NNODE = B * NSP
E = 160000
ACCROWS = 5120
MPROWS = 10240
HALF = 128

f32 = jnp.float32
i32 = jnp.int32

_NLP = pltpu.CompilerParams(needs_layout_passes=False)


def _lerp_matrix():
    ys = jnp.linspace(0.0, H - 1.0, OH)
    y0 = jnp.floor(ys).astype(i32)
    y1 = jnp.minimum(y0 + 1, H - 1)
    wy = ys - y0.astype(f32)
    r = jnp.zeros((OH, H), f32)
    rows = jnp.arange(OH)
    r = r.at[rows, y0].add(1.0 - wy)
    r = r.at[rows, y1].add(wy)
    return r


def _aux_rows():
    ii = jnp.repeat(jnp.arange(OH, dtype=f32), OH) / (OH - 1.0)
    jj = jnp.tile(jnp.arange(OH, dtype=f32), OH) / (OH - 1.0)
    a = jnp.zeros((NPIX, HALF), f32)
    return a.at[:, 0].set(1.0).at[:, 1].set(ii).at[:, 2].set(jj)


def _mm_kernel(a_ref, b_ref, o_ref):
    o_ref[...] = jnp.dot(a_ref[...], b_ref[...],
                         preferred_element_type=f32,
                         precision=lax.Precision.HIGHEST)


def _xlerp(x2):
    m = x2.shape[0]
    blk = m // 32
    return pl.pallas_call(
        _mm_kernel,
        grid=(32,),
        in_specs=[pl.BlockSpec((blk, H), lambda i: (i, 0)),
                  pl.BlockSpec((H, OH), lambda i: (0, 0))],
        out_specs=pl.BlockSpec((blk, OH), lambda i: (i, 0)),
        out_shape=jax.ShapeDtypeStruct((m, OH), f32),
    )(x2, _lerp_matrix().T)


def _transpose_kernel(a_ref, e_ref, o_ref):
    o_ref[0] = lax.dot_general(a_ref[0], e_ref[...],
                               (((0,), (0,)), ((), ())),
                               preferred_element_type=f32,
                               precision=lax.Precision.HIGHEST)


def _transpose8(a):
    k = a.shape[2]
    kb = k // 7
    return pl.pallas_call(
        _transpose_kernel,
        grid=(8, 7),
        in_specs=[pl.BlockSpec((1, C, kb), lambda i, j: (i, 0, j)),
                  pl.BlockSpec((C, C), lambda i, j: (0, 0))],
        out_specs=pl.BlockSpec((1, kb, C), lambda i, j: (i, j, 0)),
        out_shape=jax.ShapeDtypeStruct((8, k, C), f32),
    )(a, jnp.eye(C, dtype=f32))


def _ylerp_kernel(r_ref, a_ref, o_ref):
    o_ref[0] = jnp.dot(r_ref[...], a_ref[0],
                       preferred_element_type=f32,
                       precision=lax.Precision.HIGHEST)


def _ylerp(a):
    k = a.shape[2]
    kb = k // 4
    return pl.pallas_call(
        _ylerp_kernel,
        grid=(8, 4),
        in_specs=[pl.BlockSpec((OH, H), lambda i, j: (0, 0)),
                  pl.BlockSpec((1, H, kb), lambda i, j: (i, 0, j))],
        out_specs=pl.BlockSpec((1, OH, kb), lambda i, j: (i, 0, j)),
        out_shape=jax.ShapeDtypeStruct((8, OH, k), f32),
    )(_lerp_matrix(), a)


def _upsample_to_rows(skip0, skip1):
    x = jnp.concatenate([skip0.reshape(B, C, H * H),
                         skip1.reshape(B, C, H * H)], axis=0)
    x2 = x.reshape(8 * C * H, H)
    a = _xlerp(x2)
    a = a.reshape(8, C, H * OH)
    at = _transpose8(a)
    at = at.reshape(8, H, OH * C)
    u = _ylerp(at)
    return u.reshape(8, NPIX, C)


def _xw0_kernel(s_ref, a_ref, w_ref, o_ref, c_ref):
    icnt = 1.0 / jnp.maximum(a_ref[:, 0:1], 1.0)
    xw = (jnp.dot(s_ref[0, 0], w_ref[:HALF], preferred_element_type=f32,
                  precision=lax.Precision.HIGHEST)
          + jnp.dot(s_ref[0, 1], w_ref[HALF:], preferred_element_type=f32,
                    precision=lax.Precision.HIGHEST)) * icnt
    o_ref[0] = xw[:, :HALF]
    o_ref[1] = xw[:, HALF:]
    z = jnp.zeros((a_ref.shape[0], 6), f32)
    c_ref[...] = jnp.concatenate(
        [a_ref[:, 1:2] * icnt, a_ref[:, 2:3] * icnt, z], axis=1)


def _xw1_kernel(g_ref, s_ref, a_ref, b_ref, w_ref, o_ref):
    icnt = 1.0 / jnp.maximum(a_ref[:, 0:1], 1.0)
    g = jnp.concatenate([g_ref[0], g_ref[1]], axis=1)
    s = jnp.concatenate([s_ref[0, 0], s_ref[0, 1]], axis=1)
    x2 = ((1.0 - MIX) * jnp.maximum(g + b_ref[...], 0.0)
          + MIX * icnt * s)
    xw = jnp.dot(x2, w_ref[...], preferred_element_type=f32,
                 precision=lax.Precision.HIGHEST)
    o_ref[0] = xw[:, :HALF]
    o_ref[1] = xw[:, HALF:]


def _y_kernel(g_ref, b_ref, wl_ref, o_ref):
    g = jnp.concatenate([g_ref[0], g_ref[1]], axis=1)
    x3 = jnp.maximum(g + b_ref[...], 0.0)
    o_ref[...] = jnp.sum(x3 * wl_ref[...], axis=1, keepdims=True)


_RB = 2000


def _compute_xw0(ssum, asum, w0):
    return pl.pallas_call(
        _xw0_kernel,
        grid=(NNODE // _RB,),
        in_specs=[pl.BlockSpec((1, 2, _RB, HALF), lambda i: (0, 0, i, 0)),
                  pl.BlockSpec((_RB, HALF), lambda i: (i, 0)),
                  pl.BlockSpec((C, C), lambda i: (0, 0))],
        out_specs=(pl.BlockSpec((2, _RB, HALF), lambda i: (0, i, 0)),
                   pl.BlockSpec((_RB, 8), lambda i: (i, 0))),
        out_shape=(jax.ShapeDtypeStruct((2, NNODE, HALF), f32),
                   jax.ShapeDtypeStruct((NNODE, 8), f32)),
    )(ssum, asum, w0)


def _compute_xw1(agg0, ssum, asum, b0, w1):
    return pl.pallas_call(
        _xw1_kernel,
        grid=(NNODE // _RB,),
        in_specs=[pl.BlockSpec((2, _RB, HALF), lambda i: (0, i, 0)),
                  pl.BlockSpec((1, 2, _RB, HALF), lambda i: (1, 0, i, 0)),
                  pl.BlockSpec((_RB, HALF), lambda i: (i, 0)),
                  pl.BlockSpec((1, C), lambda i: (0, 0)),
                  pl.BlockSpec((C, C), lambda i: (0, 0))],
        out_specs=pl.BlockSpec((2, _RB, HALF), lambda i: (0, i, 0)),
        out_shape=jax.ShapeDtypeStruct((2, NNODE, HALF), f32),
    )(agg0, ssum, asum, b0.reshape(1, C), w1)


def _compute_y(agg1, b1, w_lin):
    return pl.pallas_call(
        _y_kernel,
        grid=(NNODE // _RB,),
        in_specs=[pl.BlockSpec((2, _RB, HALF), lambda i: (0, i, 0)),
                  pl.BlockSpec((1, C), lambda i: (0, 0)),
                  pl.BlockSpec((1, C), lambda i: (0, 0))],
        out_specs=pl.BlockSpec((_RB, 1), lambda i: (i, 0)),
        out_shape=jax.ShapeDtypeStruct((NNODE, 1), f32),
    )(agg1, b1.reshape(1, C), w_lin.reshape(1, C))


@functools.cache
def _sc_mesh():
    return plsc.VectorSubcoreMesh(core_axis_name="c", subcore_axis_name="s")


_PCH = 112
_PPT = NPIX // 16
_ZR = ACCROWS // 16


def _pool_body(u_hbm, lab_hbm, aux_hbm, zacc_hbm, ssum_hbm, asum_hbm,
               acc, lab0, row0, gs0, ss0, lab1, row1, gs1, ss1):
    cid = lax.axis_index("c")
    sid = lax.axis_index("s")
    bufs = ((lab0, row0, gs0, ss0), (lab1, row1, gs1, ss1))
    gdesc = [None, None]
    sdesc = [None, None]

    def fetch(chunk, b):
        img, src_at, base = chunk
        labv, rows, gs, _ = bufs[b]
        if sdesc[b] is not None:
            sdesc[b].wait()
            sdesc[b] = None
        pltpu.sync_copy(lab_hbm.at[pl.ds(img * NPIX + base, _PCH)], labv)
        gdesc[b] = pltpu.async_copy(src_at(base), rows, gs)

    def scatter(b):
        labv, rows, _, ss = bufs[b]
        gdesc[b].wait()
        gdesc[b] = None
        sdesc[b] = pltpu.async_copy(rows, acc.at[labv], ss, add=True)

    def run_phase(chunks):
        fetch(chunks[0], 0)
        for i in range(len(chunks)):
            if i + 1 < len(chunks):
                fetch(chunks[i + 1], (i + 1) % 2)
            scatter(i % 2)
        for b in range(2):
            if sdesc[b] is not None:
                sdesc[b].wait()
                sdesc[b] = None

    def drain(out_at):
        pltpu.sync_copy(acc.at[pl.ds(sid * 312, 312)],
                        out_at(sid * 312, 312))

        @pl.when(sid == 0)
        def _():
            pltpu.sync_copy(acc.at[pl.ds(4992, 8)], out_at(4992, 8))

    def phase_chunks(src_for_img):
        chunks = []
        for bl in range(2):
            img = cid * 2 + bl
            src_at = src_for_img(bl)
            for k in range(_PPT // _PCH):
                chunks.append((img, src_at, sid * _PPT + k * _PCH))
        return chunks

    for m in range(2):
        for h in range(2):
            pltpu.sync_copy(zacc_hbm.at[pl.ds(sid * _ZR, _ZR)],
                            acc.at[pl.ds(sid * _ZR, _ZR)])
            plsc.subcore_barrier()

            def usrc(bl, m=m, h=h):
                mi = m * 4 + cid * 2 + bl
                return lambda base: u_hbm.at[mi, pl.ds(base, _PCH), h]

            run_phase(phase_chunks(usrc))
            plsc.subcore_barrier()
            drain(lambda r, n: ssum_hbm.at[m, h, pl.ds(cid * 5000 + r, n)])
            plsc.subcore_barrier()

    pltpu.sync_copy(zacc_hbm.at[pl.ds(sid * _ZR, _ZR)],
                    acc.at[pl.ds(sid * _ZR, _ZR)])
    plsc.subcore_barrier()
    run_phase(phase_chunks(
        lambda bl: lambda base: aux_hbm.at[pl.ds(base, _PCH)]))
    plsc.subcore_barrier()
    drain(lambda r, n: asum_hbm.at[pl.ds(cid * 5000 + r, n)])


def _sc_pool(u4, labels2, aux, zacc):
    fn = pl.kernel(
        _pool_body,
        out_type=(jax.ShapeDtypeStruct((2, 2, NNODE, HALF), f32),
                  jax.ShapeDtypeStruct((NNODE, HALF), f32)),
        mesh=_sc_mesh(),
        scratch_types=[
            pltpu.VMEM_SHARED((ACCROWS, HALF), f32),
            pltpu.VMEM((_PCH,), i32),
            pltpu.VMEM((_PCH, HALF), f32),
            pltpu.SemaphoreType.DMA,
            pltpu.SemaphoreType.DMA,
            pltpu.VMEM((_PCH,), i32),
            pltpu.VMEM((_PCH, HALF), f32),
            pltpu.SemaphoreType.DMA,
            pltpu.SemaphoreType.DMA,
        ],
        compiler_params=_NLP,
    )
    return fn(u4, labels2, aux, zacc)


_EPW = E // 32
_EVR = _EPW // 16


def _sanitize_tail(ref):
    lanes = lax.iota(i32, 16)
    v = ref[pl.ds(4992, 16)]
    ref[pl.ds(4992, 16)] = jnp.where(lanes < 8, v, 0)


def _eattr_body(e0_hbm, e1_hbm, tab_hbm, ea_hbm, tab, e0v, e1v, outv):
    cid = lax.axis_index("c")
    sid = lax.axis_index("s")
    wid = sid * 2 + cid
    base = wid * _EPW
    pltpu.sync_copy(tab_hbm, tab)
    pltpu.sync_copy(e0_hbm.at[pl.ds(base, _EPW)], e0v.at[pl.ds(0, _EPW)])
    pltpu.sync_copy(e1_hbm.at[pl.ds(base, _EPW)], e1v.at[pl.ds(0, _EPW)])
    _sanitize_tail(e0v)
    _sanitize_tail(e1v)

    def body(i, carry):
        sl = pl.ds(i * 16, 16)
        a = e0v[sl] * 8
        b = e1v[sl] * 8
        dx = plsc.load_gather(tab, [a]) - plsc.load_gather(tab, [b])
        dy = plsc.load_gather(tab, [a + 1]) - plsc.load_gather(tab, [b + 1])
        outv[sl] = jnp.exp(-(dx * dx + dy * dy) * (1.0 / SIGMA))
        return carry

    lax.fori_loop(0, _EVR + 1, body, 0)
    pltpu.sync_copy(outv.at[pl.ds(0, _EPW)], ea_hbm.at[pl.ds(base, _EPW)])


def _sc_edge_attr(e0, e1, tabflat):
    fn = pl.kernel(
        _eattr_body,
        out_type=jax.ShapeDtypeStruct((E,), f32),
        mesh=_sc_mesh(),
        scratch_types=[
            pltpu.VMEM((NNODE * 8,), f32),
            pltpu.VMEM((_EPW + 16,), i32),
            pltpu.VMEM((_EPW + 16,), i32),
            pltpu.VMEM((_EPW + 16,), f32),
        ],
        compiler_params=_NLP,
    )
    return fn(e0, e1, tabflat)


_ECH = 80
_EPT = E // 16
_NCH = _EPT // _ECH


def _mp_body(xw_hbm, e0_hbm, e1_hbm, ea_hbm, zmp_hbm, agg_hbm, acc,
             s0, d0, w0v, r0, gs0, ss0, s1, d1, w1v, r1, gs1, ss1,
             s2, d2, w2v, r2, gs2, ss2):
    cid = lax.axis_index("c")
    sid = lax.axis_index("s")
    pltpu.sync_copy(zmp_hbm.at[pl.ds(sid * 640, 640)],
                    acc.at[pl.ds(sid * 640, 640)])
    plsc.subcore_barrier()
    bufs = ((s0, d0, w0v, r0, gs0, ss0), (s1, d1, w1v, r1, gs1, ss1),
            (s2, d2, w2v, r2, gs2, ss2))

    def fetch(k, b):
        sv, dv, wv, rw, gs, _ = bufs[b]
        base = sid * _EPT + k * _ECH
        pltpu.sync_copy(e0_hbm.at[pl.ds(base, _ECH)], sv)
        pltpu.sync_copy(e1_hbm.at[pl.ds(base, _ECH)], dv)
        pltpu.sync_copy(ea_hbm.at[pl.ds(base, _ECH)], wv)
        for t in range(_ECH // 16):
            sv[pl.ds(t * 16, 16)] = sv[pl.ds(t * 16, 16)] + cid * NNODE
        pltpu.async_copy(xw_hbm.at[sv], rw, gs)

    def compute(b):
        sv, dv, wv, rw, gs, ss = bufs[b]
        pltpu.make_async_copy(xw_hbm.at[sv], rw, gs).wait()

        def edge(j, c2):
            s = plsc.load_gather(wv, [jnp.full((16,), 0, i32) + j])
            for q in range(HALF // 16):
                sl = pl.ds(q * 16, 16)
                rw[j, sl] = rw[j, sl] * s
            return c2

        lax.fori_loop(0, _ECH, edge, 0)
        pltpu.async_copy(rw, acc.at[dv], ss, add=True)

    def wait_scatter(b):
        _, dv, _, rw, _, ss = bufs[b]
        pltpu.make_async_copy(rw, acc.at[dv], ss).wait()

    def step(c, b, f, wait_sc):
        if f is not None:
            if wait_sc:
                wait_scatter(f % 3)
            fetch(f, f % 3)
        compute(b)

    fetch(0, 0)
    fetch(1, 1)
    step(0, 0, 2, False)
    step(1, 1, 3, True)
    step(2, 2, 4, True)

    def k3body(k3, carry):
        c = 3 * k3
        for j in range(3):
            wait_scatter((j + 2) % 3)
            fetch(c + j + 2, (j + 2) % 3)
            compute(j)
        return carry

    lax.fori_loop(1, (_NCH - 5) // 3 + 1, k3body, 0)
    compute(0)
    compute(1)
    wait_scatter(0)
    wait_scatter(1)
    wait_scatter(2)
    plsc.subcore_barrier()
    pltpu.sync_copy(acc.at[pl.ds(sid * 624, 624)],
                    agg_hbm.at[cid, pl.ds(sid * 624, 624)])

    @pl.when(sid == 0)
    def _():
        pltpu.sync_copy(acc.at[pl.ds(9984, 16)],
                        agg_hbm.at[cid, pl.ds(9984, 16)])


def _sc_message_pass(xwflat, e0, e1, ea, zmp):
    fn = pl.kernel(
        _mp_body,
        out_type=jax.ShapeDtypeStruct((2, NNODE, HALF), f32),
        mesh=_sc_mesh(),
        scratch_types=[
            pltpu.VMEM_SHARED((MPROWS, HALF), f32),
            pltpu.VMEM((_ECH,), i32),
            pltpu.VMEM((_ECH,), i32),
            pltpu.VMEM((_ECH,), f32),
            pltpu.VMEM((_ECH, HALF), f32),
            pltpu.SemaphoreType.DMA,
            pltpu.SemaphoreType.DMA,
            pltpu.VMEM((_ECH,), i32),
            pltpu.VMEM((_ECH,), i32),
            pltpu.VMEM((_ECH,), f32),
            pltpu.VMEM((_ECH, HALF), f32),
            pltpu.SemaphoreType.DMA,
            pltpu.SemaphoreType.DMA,
            pltpu.VMEM((_ECH,), i32),
            pltpu.VMEM((_ECH,), i32),
            pltpu.VMEM((_ECH,), f32),
            pltpu.VMEM((_ECH, HALF), f32),
            pltpu.SemaphoreType.DMA,
            pltpu.SemaphoreType.DMA,
        ],
        compiler_params=_NLP,
    )
    return fn(xwflat, e0, e1, ea, zmp)


def _scores_body(y_hbm, e0_hbm, e1_hbm, negs_hbm, dan_hbm, dap_hbm,
                 yv, e0v, e1v, env, danv, dapv):
    cid = lax.axis_index("c")
    sid = lax.axis_index("s")
    wid = sid * 2 + cid
    base = wid * _EPW
    pltpu.sync_copy(y_hbm, yv)
    pltpu.sync_copy(e0_hbm.at[pl.ds(base, _EPW)], e0v.at[pl.ds(0, _EPW)])
    pltpu.sync_copy(e1_hbm.at[pl.ds(base, _EPW)], e1v.at[pl.ds(0, _EPW)])
    pltpu.sync_copy(negs_hbm.at[pl.ds(base, _EPW)], env.at[pl.ds(0, _EPW)])
    _sanitize_tail(e0v)
    _sanitize_tail(e1v)
    _sanitize_tail(env)

    def body(i, carry):
        sl = pl.ds(i * 16, 16)
        s0 = plsc.load_gather(yv, [e0v[sl]])
        s1 = plsc.load_gather(yv, [e1v[sl]])
        sn = plsc.load_gather(yv, [env[sl]])
        dapv[sl] = 1.0 / (1.0 + jnp.exp(s1 - s0))
        danv[sl] = 1.0 / (1.0 + jnp.exp(sn - s0))
        return carry

    lax.fori_loop(0, _EVR + 1, body, 0)
    pltpu.sync_copy(danv.at[pl.ds(0, _EPW)], dan_hbm.at[pl.ds(base, _EPW)])
    pltpu.sync_copy(dapv.at[pl.ds(0, _EPW)], dap_hbm.at[pl.ds(base, _EPW)])


def _sc_scores(y, e0, e1, negs):
    fn = pl.kernel(
        _scores_body,
        out_type=(jax.ShapeDtypeStruct((E,), f32),
                  jax.ShapeDtypeStruct((E,), f32)),
        mesh=_sc_mesh(),
        scratch_types=[
            pltpu.VMEM((NNODE,), f32),
            pltpu.VMEM((_EPW + 16,), i32),
            pltpu.VMEM((_EPW + 16,), i32),
            pltpu.VMEM((_EPW + 16,), i32),
            pltpu.VMEM((_EPW + 16,), f32),
            pltpu.VMEM((_EPW + 16,), f32),
        ],
        compiler_params=_NLP,
    )
    return fn(y, e0, e1, negs)


def kernel(labels, autoenc_skip0, autoenc_skip1, edges_nn, negs,
           W0, b0, W1, b1, w_lin):
    off = jnp.array([0, NSP, 0, NSP], i32).reshape(B, 1)
    labels2 = (labels.reshape(B, NPIX).astype(i32) + off).reshape(B * NPIX)
    e0 = edges_nn[0].astype(i32)
    e1 = edges_nn[1].astype(i32)
    negs = negs.astype(i32)

    u = _upsample_to_rows(autoenc_skip0, autoenc_skip1)
    u4 = u.reshape(8, NPIX, 2, HALF)

    zacc = jnp.zeros((ACCROWS, HALF), f32)
    zmp = jnp.zeros((MPROWS, HALF), f32)

    ssum, asum = _sc_pool(u4, labels2, _aux_rows(), zacc)
    xw0, coords = _compute_xw0(ssum, asum, W0)
    ea = _sc_edge_attr(e0, e1, coords.reshape(NNODE * 8))

    agg0 = _sc_message_pass(xw0.reshape(2 * NNODE, HALF), e0, e1, ea, zmp)
    xw1 = _compute_xw1(agg0, ssum, asum, b0, W1)
    agg1 = _sc_message_pass(xw1.reshape(2 * NNODE, HALF), e0, e1, ea, zmp)
    y = _compute_y(agg1, b1, w_lin)

    dan, dap = _sc_scores(y.reshape(NNODE), e0, e1, negs)
    return (dan.reshape(E, 1), dap.reshape(E, 1), ea)


# speedup vs baseline: 3.3554x; 1.0484x over previous
import functools

import jax
import jax.numpy as jnp
from jax import lax
from jax.experimental import pallas as pl
from jax.experimental.pallas import tpu as pltpu
from jax.experimental.pallas import tpu_sc as plsc

SIGMA = 0.05
MIX = 0.5
NSP = 2500
B = 4
C = 256
H = 56
OH = 112
NPIX = OH * OH
NNODE = B * NSP
E = 160000
ACCROWS = 5120
MPROWS = 10240
HALF = 128

f32 = jnp.float32
i32 = jnp.int32

_NLP = pltpu.CompilerParams(needs_layout_passes=False)


def _lerp_matrix():
    ys = jnp.linspace(0.0, H - 1.0, OH)
    y0 = jnp.floor(ys).astype(i32)
    y1 = jnp.minimum(y0 + 1, H - 1)
    wy = ys - y0.astype(f32)
    r = jnp.zeros((OH, H), f32)
    rows = jnp.arange(OH)
    r = r.at[rows, y0].add(1.0 - wy)
    r = r.at[rows, y1].add(wy)
    return r


def _aux_rows():
    ii = jnp.repeat(jnp.arange(OH, dtype=f32), OH) / (OH - 1.0)
    jj = jnp.tile(jnp.arange(OH, dtype=f32), OH) / (OH - 1.0)
    a = jnp.zeros((NPIX, HALF), f32)
    return a.at[:, 0].set(1.0).at[:, 1].set(ii).at[:, 2].set(jj)


def _mm_kernel(a_ref, b_ref, o_ref):
    o_ref[...] = jnp.dot(a_ref[...], b_ref[...],
                         preferred_element_type=f32,
                         precision=lax.Precision.DEFAULT)


def _xlerp(x2):
    m = x2.shape[0]
    blk = m // 32
    return pl.pallas_call(
        _mm_kernel,
        grid=(32,),
        in_specs=[pl.BlockSpec((blk, H), lambda i: (i, 0)),
                  pl.BlockSpec((H, OH), lambda i: (0, 0))],
        out_specs=pl.BlockSpec((blk, OH), lambda i: (i, 0)),
        out_shape=jax.ShapeDtypeStruct((m, OH), f32),
    )(x2, _lerp_matrix().T)


def _transpose_kernel(a_ref, e_ref, o_ref):
    o_ref[0] = lax.dot_general(a_ref[0], e_ref[...],
                               (((0,), (0,)), ((), ())),
                               preferred_element_type=f32,
                               precision=lax.Precision.DEFAULT)


def _transpose8(a):
    k = a.shape[2]
    kb = k // 7
    return pl.pallas_call(
        _transpose_kernel,
        grid=(8, 7),
        in_specs=[pl.BlockSpec((1, C, kb), lambda i, j: (i, 0, j)),
                  pl.BlockSpec((C, C), lambda i, j: (0, 0))],
        out_specs=pl.BlockSpec((1, kb, C), lambda i, j: (i, j, 0)),
        out_shape=jax.ShapeDtypeStruct((8, k, C), f32),
    )(a, jnp.eye(C, dtype=f32))


def _ylerp_kernel(r_ref, a_ref, o_ref):
    o_ref[0] = jnp.dot(r_ref[...], a_ref[0],
                       preferred_element_type=f32,
                       precision=lax.Precision.DEFAULT)


def _ylerp(a):
    k = a.shape[2]
    kb = k // 4
    return pl.pallas_call(
        _ylerp_kernel,
        grid=(8, 4),
        in_specs=[pl.BlockSpec((OH, H), lambda i, j: (0, 0)),
                  pl.BlockSpec((1, H, kb), lambda i, j: (i, 0, j))],
        out_specs=pl.BlockSpec((1, OH, kb), lambda i, j: (i, 0, j)),
        out_shape=jax.ShapeDtypeStruct((8, OH, k), f32),
    )(_lerp_matrix(), a)


def _upsample_to_rows(skip0, skip1):
    x = jnp.concatenate([skip0.reshape(B, C, H * H),
                         skip1.reshape(B, C, H * H)], axis=0)
    x2 = x.reshape(8 * C * H, H)
    a = _xlerp(x2)
    a = a.reshape(8, C, H * OH)
    at = _transpose8(a)
    at = at.reshape(8, H, OH * C)
    u = _ylerp(at)
    return u.reshape(8, NPIX, C)


def _xw0_kernel(s_ref, a_ref, w_ref, o_ref, c_ref):
    icnt = 1.0 / jnp.maximum(a_ref[:, 0:1], 1.0)
    xw = (jnp.dot(s_ref[0, 0], w_ref[:HALF], preferred_element_type=f32,
                  precision=lax.Precision.DEFAULT)
          + jnp.dot(s_ref[0, 1], w_ref[HALF:], preferred_element_type=f32,
                    precision=lax.Precision.DEFAULT)) * icnt
    o_ref[0] = xw[:, :HALF]
    o_ref[1] = xw[:, HALF:]
    z = jnp.zeros((a_ref.shape[0], 6), f32)
    c_ref[...] = jnp.concatenate(
        [a_ref[:, 1:2] * icnt, a_ref[:, 2:3] * icnt, z], axis=1)


def _xw1_kernel(g_ref, s_ref, a_ref, b_ref, w_ref, o_ref):
    icnt = 1.0 / jnp.maximum(a_ref[:, 0:1], 1.0)
    g = jnp.concatenate([g_ref[0], g_ref[1]], axis=1)
    s = jnp.concatenate([s_ref[0, 0], s_ref[0, 1]], axis=1)
    x2 = ((1.0 - MIX) * jnp.maximum(g + b_ref[...], 0.0)
          + MIX * icnt * s)
    xw = jnp.dot(x2, w_ref[...], preferred_element_type=f32,
                 precision=lax.Precision.DEFAULT)
    o_ref[0] = xw[:, :HALF]
    o_ref[1] = xw[:, HALF:]


def _y_kernel(g_ref, b_ref, wl_ref, o_ref):
    g = jnp.concatenate([g_ref[0], g_ref[1]], axis=1)
    x3 = jnp.maximum(g + b_ref[...], 0.0)
    o_ref[...] = jnp.sum(x3 * wl_ref[...], axis=1, keepdims=True)


_RB = 2000


def _compute_xw0(ssum, asum, w0):
    return pl.pallas_call(
        _xw0_kernel,
        grid=(NNODE // _RB,),
        in_specs=[pl.BlockSpec((1, 2, _RB, HALF), lambda i: (0, 0, i, 0)),
                  pl.BlockSpec((_RB, HALF), lambda i: (i, 0)),
                  pl.BlockSpec((C, C), lambda i: (0, 0))],
        out_specs=(pl.BlockSpec((2, _RB, HALF), lambda i: (0, i, 0)),
                   pl.BlockSpec((_RB, 8), lambda i: (i, 0))),
        out_shape=(jax.ShapeDtypeStruct((2, NNODE, HALF), f32),
                   jax.ShapeDtypeStruct((NNODE, 8), f32)),
    )(ssum, asum, w0)


def _compute_xw1(agg0, ssum, asum, b0, w1):
    return pl.pallas_call(
        _xw1_kernel,
        grid=(NNODE // _RB,),
        in_specs=[pl.BlockSpec((2, _RB, HALF), lambda i: (0, i, 0)),
                  pl.BlockSpec((1, 2, _RB, HALF), lambda i: (1, 0, i, 0)),
                  pl.BlockSpec((_RB, HALF), lambda i: (i, 0)),
                  pl.BlockSpec((1, C), lambda i: (0, 0)),
                  pl.BlockSpec((C, C), lambda i: (0, 0))],
        out_specs=pl.BlockSpec((2, _RB, HALF), lambda i: (0, i, 0)),
        out_shape=jax.ShapeDtypeStruct((2, NNODE, HALF), f32),
    )(agg0, ssum, asum, b0.reshape(1, C), w1)


def _compute_y(agg1, b1, w_lin):
    return pl.pallas_call(
        _y_kernel,
        grid=(NNODE // _RB,),
        in_specs=[pl.BlockSpec((2, _RB, HALF), lambda i: (0, i, 0)),
                  pl.BlockSpec((1, C), lambda i: (0, 0)),
                  pl.BlockSpec((1, C), lambda i: (0, 0))],
        out_specs=pl.BlockSpec((_RB, 1), lambda i: (i, 0)),
        out_shape=jax.ShapeDtypeStruct((NNODE, 1), f32),
    )(agg1, b1.reshape(1, C), w_lin.reshape(1, C))


@functools.cache
def _sc_mesh():
    return plsc.VectorSubcoreMesh(core_axis_name="c", subcore_axis_name="s")


_PCH = 112
_PPT = NPIX // 16
_ZR = ACCROWS // 16


def _pool_body(u_hbm, lab_hbm, aux_hbm, zacc_hbm, ssum_hbm, asum_hbm,
               acc, lab0, row0, gs0, ss0, lab1, row1, gs1, ss1):
    cid = lax.axis_index("c")
    sid = lax.axis_index("s")
    bufs = ((lab0, row0, gs0, ss0), (lab1, row1, gs1, ss1))
    gdesc = [None, None]
    sdesc = [None, None]

    def fetch(chunk, b):
        img, src_at, base = chunk
        labv, rows, gs, _ = bufs[b]
        if sdesc[b] is not None:
            sdesc[b].wait()
            sdesc[b] = None
        pltpu.sync_copy(lab_hbm.at[pl.ds(img * NPIX + base, _PCH)], labv)
        gdesc[b] = pltpu.async_copy(src_at(base), rows, gs)

    def scatter(b):
        labv, rows, _, ss = bufs[b]
        gdesc[b].wait()
        gdesc[b] = None
        sdesc[b] = pltpu.async_copy(rows, acc.at[labv], ss, add=True)

    def run_phase(chunks):
        fetch(chunks[0], 0)
        for i in range(len(chunks)):
            if i + 1 < len(chunks):
                fetch(chunks[i + 1], (i + 1) % 2)
            scatter(i % 2)
        for b in range(2):
            if sdesc[b] is not None:
                sdesc[b].wait()
                sdesc[b] = None

    def drain(out_at):
        pltpu.sync_copy(acc.at[pl.ds(sid * 312, 312)],
                        out_at(sid * 312, 312))

        @pl.when(sid == 0)
        def _():
            pltpu.sync_copy(acc.at[pl.ds(4992, 8)], out_at(4992, 8))

    def phase_chunks(src_for_img):
        chunks = []
        for bl in range(2):
            img = cid * 2 + bl
            src_at = src_for_img(bl)
            for k in range(_PPT // _PCH):
                chunks.append((img, src_at, sid * _PPT + k * _PCH))
        return chunks

    for m in range(2):
        for h in range(2):
            pltpu.sync_copy(zacc_hbm.at[pl.ds(sid * _ZR, _ZR)],
                            acc.at[pl.ds(sid * _ZR, _ZR)])
            plsc.subcore_barrier()

            def usrc(bl, m=m, h=h):
                mi = m * 4 + cid * 2 + bl
                return lambda base: u_hbm.at[mi, pl.ds(base, _PCH), h]

            run_phase(phase_chunks(usrc))
            plsc.subcore_barrier()
            drain(lambda r, n: ssum_hbm.at[m, h, pl.ds(cid * 5000 + r, n)])
            plsc.subcore_barrier()

    pltpu.sync_copy(zacc_hbm.at[pl.ds(sid * _ZR, _ZR)],
                    acc.at[pl.ds(sid * _ZR, _ZR)])
    plsc.subcore_barrier()
    run_phase(phase_chunks(
        lambda bl: lambda base: aux_hbm.at[pl.ds(base, _PCH)]))
    plsc.subcore_barrier()
    drain(lambda r, n: asum_hbm.at[pl.ds(cid * 5000 + r, n)])


def _sc_pool(u4, labels2, aux, zacc):
    fn = pl.kernel(
        _pool_body,
        out_type=(jax.ShapeDtypeStruct((2, 2, NNODE, HALF), f32),
                  jax.ShapeDtypeStruct((NNODE, HALF), f32)),
        mesh=_sc_mesh(),
        scratch_types=[
            pltpu.VMEM_SHARED((ACCROWS, HALF), f32),
            pltpu.VMEM((_PCH,), i32),
            pltpu.VMEM((_PCH, HALF), f32),
            pltpu.SemaphoreType.DMA,
            pltpu.SemaphoreType.DMA,
            pltpu.VMEM((_PCH,), i32),
            pltpu.VMEM((_PCH, HALF), f32),
            pltpu.SemaphoreType.DMA,
            pltpu.SemaphoreType.DMA,
        ],
        compiler_params=_NLP,
    )
    return fn(u4, labels2, aux, zacc)


_EPW = E // 32
_EVR = _EPW // 16


def _sanitize_tail(ref):
    lanes = lax.iota(i32, 16)
    v = ref[pl.ds(4992, 16)]
    ref[pl.ds(4992, 16)] = jnp.where(lanes < 8, v, 0)


def _eattr_body(e0_hbm, e1_hbm, tab_hbm, ea_hbm, tab, e0v, e1v, outv):
    cid = lax.axis_index("c")
    sid = lax.axis_index("s")
    wid = sid * 2 + cid
    base = wid * _EPW
    pltpu.sync_copy(tab_hbm, tab)
    pltpu.sync_copy(e0_hbm.at[pl.ds(base, _EPW)], e0v.at[pl.ds(0, _EPW)])
    pltpu.sync_copy(e1_hbm.at[pl.ds(base, _EPW)], e1v.at[pl.ds(0, _EPW)])
    _sanitize_tail(e0v)
    _sanitize_tail(e1v)

    def body(i, carry):
        sl = pl.ds(i * 16, 16)
        a = e0v[sl] * 8
        b = e1v[sl] * 8
        dx = plsc.load_gather(tab, [a]) - plsc.load_gather(tab, [b])
        dy = plsc.load_gather(tab, [a + 1]) - plsc.load_gather(tab, [b + 1])
        outv[sl] = jnp.exp(-(dx * dx + dy * dy) * (1.0 / SIGMA))
        return carry

    lax.fori_loop(0, _EVR + 1, body, 0)
    pltpu.sync_copy(outv.at[pl.ds(0, _EPW)], ea_hbm.at[pl.ds(base, _EPW)])


def _sc_edge_attr(e0, e1, tabflat):
    fn = pl.kernel(
        _eattr_body,
        out_type=jax.ShapeDtypeStruct((E,), f32),
        mesh=_sc_mesh(),
        scratch_types=[
            pltpu.VMEM((NNODE * 8,), f32),
            pltpu.VMEM((_EPW + 16,), i32),
            pltpu.VMEM((_EPW + 16,), i32),
            pltpu.VMEM((_EPW + 16,), f32),
        ],
        compiler_params=_NLP,
    )
    return fn(e0, e1, tabflat)


_ECH = 80
_EPT = E // 16
_NCH = _EPT // _ECH


def _mp_body(xw_hbm, e0_hbm, e1_hbm, ea_hbm, zmp_hbm, agg_hbm, acc,
             s0, d0, w0v, r0, gs0, ss0, s1, d1, w1v, r1, gs1, ss1,
             s2, d2, w2v, r2, gs2, ss2):
    cid = lax.axis_index("c")
    sid = lax.axis_index("s")
    pltpu.sync_copy(zmp_hbm.at[pl.ds(sid * 640, 640)],
                    acc.at[pl.ds(sid * 640, 640)])
    plsc.subcore_barrier()
    bufs = ((s0, d0, w0v, r0, gs0, ss0), (s1, d1, w1v, r1, gs1, ss1),
            (s2, d2, w2v, r2, gs2, ss2))

    def fetch(k, b):
        sv, dv, wv, rw, gs, _ = bufs[b]
        base = sid * _EPT + k * _ECH
        pltpu.sync_copy(e0_hbm.at[pl.ds(base, _ECH)], sv)
        pltpu.sync_copy(e1_hbm.at[pl.ds(base, _ECH)], dv)
        pltpu.sync_copy(ea_hbm.at[pl.ds(base, _ECH)], wv)
        for t in range(_ECH // 16):
            sv[pl.ds(t * 16, 16)] = sv[pl.ds(t * 16, 16)] + cid * NNODE
        pltpu.async_copy(xw_hbm.at[sv], rw, gs)

    def compute(b):
        sv, dv, wv, rw, gs, ss = bufs[b]
        pltpu.make_async_copy(xw_hbm.at[sv], rw, gs).wait()

        def edge(j, c2):
            s = plsc.load_gather(wv, [jnp.full((16,), 0, i32) + j])
            for q in range(HALF // 16):
                sl = pl.ds(q * 16, 16)
                rw[j, sl] = rw[j, sl] * s
            return c2

        lax.fori_loop(0, _ECH, edge, 0)
        pltpu.async_copy(rw, acc.at[dv], ss, add=True)

    def wait_scatter(b):
        _, dv, _, rw, _, ss = bufs[b]
        pltpu.make_async_copy(rw, acc.at[dv], ss).wait()

    def step(c, b, f, wait_sc):
        if f is not None:
            if wait_sc:
                wait_scatter(f % 3)
            fetch(f, f % 3)
        compute(b)

    fetch(0, 0)
    fetch(1, 1)
    step(0, 0, 2, False)
    step(1, 1, 3, True)
    step(2, 2, 4, True)

    def k3body(k3, carry):
        c = 3 * k3
        for j in range(3):
            wait_scatter((j + 2) % 3)
            fetch(c + j + 2, (j + 2) % 3)
            compute(j)
        return carry

    lax.fori_loop(1, (_NCH - 5) // 3 + 1, k3body, 0)
    compute(0)
    compute(1)
    wait_scatter(0)
    wait_scatter(1)
    wait_scatter(2)
    plsc.subcore_barrier()
    pltpu.sync_copy(acc.at[pl.ds(sid * 624, 624)],
                    agg_hbm.at[cid, pl.ds(sid * 624, 624)])

    @pl.when(sid == 0)
    def _():
        pltpu.sync_copy(acc.at[pl.ds(9984, 16)],
                        agg_hbm.at[cid, pl.ds(9984, 16)])


def _sc_message_pass(xwflat, e0, e1, ea, zmp):
    fn = pl.kernel(
        _mp_body,
        out_type=jax.ShapeDtypeStruct((2, NNODE, HALF), f32),
        mesh=_sc_mesh(),
        scratch_types=[
            pltpu.VMEM_SHARED((MPROWS, HALF), f32),
            pltpu.VMEM((_ECH,), i32),
            pltpu.VMEM((_ECH,), i32),
            pltpu.VMEM((_ECH,), f32),
            pltpu.VMEM((_ECH, HALF), f32),
            pltpu.SemaphoreType.DMA,
            pltpu.SemaphoreType.DMA,
            pltpu.VMEM((_ECH,), i32),
            pltpu.VMEM((_ECH,), i32),
            pltpu.VMEM((_ECH,), f32),
            pltpu.VMEM((_ECH, HALF), f32),
            pltpu.SemaphoreType.DMA,
            pltpu.SemaphoreType.DMA,
            pltpu.VMEM((_ECH,), i32),
            pltpu.VMEM((_ECH,), i32),
            pltpu.VMEM((_ECH,), f32),
            pltpu.VMEM((_ECH, HALF), f32),
            pltpu.SemaphoreType.DMA,
            pltpu.SemaphoreType.DMA,
        ],
        compiler_params=_NLP,
    )
    return fn(xwflat, e0, e1, ea, zmp)


def _scores_body(y_hbm, e0_hbm, e1_hbm, negs_hbm, dan_hbm, dap_hbm,
                 yv, e0v, e1v, env, danv, dapv):
    cid = lax.axis_index("c")
    sid = lax.axis_index("s")
    wid = sid * 2 + cid
    base = wid * _EPW
    pltpu.sync_copy(y_hbm, yv)
    pltpu.sync_copy(e0_hbm.at[pl.ds(base, _EPW)], e0v.at[pl.ds(0, _EPW)])
    pltpu.sync_copy(e1_hbm.at[pl.ds(base, _EPW)], e1v.at[pl.ds(0, _EPW)])
    pltpu.sync_copy(negs_hbm.at[pl.ds(base, _EPW)], env.at[pl.ds(0, _EPW)])
    _sanitize_tail(e0v)
    _sanitize_tail(e1v)
    _sanitize_tail(env)

    def body(i, carry):
        sl = pl.ds(i * 16, 16)
        s0 = plsc.load_gather(yv, [e0v[sl]])
        s1 = plsc.load_gather(yv, [e1v[sl]])
        sn = plsc.load_gather(yv, [env[sl]])
        dapv[sl] = 1.0 / (1.0 + jnp.exp(s1 - s0))
        danv[sl] = 1.0 / (1.0 + jnp.exp(sn - s0))
        return carry

    lax.fori_loop(0, _EVR + 1, body, 0)
    pltpu.sync_copy(danv.at[pl.ds(0, _EPW)], dan_hbm.at[pl.ds(base, _EPW)])
    pltpu.sync_copy(dapv.at[pl.ds(0, _EPW)], dap_hbm.at[pl.ds(base, _EPW)])


def _sc_scores(y, e0, e1, negs):
    fn = pl.kernel(
        _scores_body,
        out_type=(jax.ShapeDtypeStruct((E,), f32),
                  jax.ShapeDtypeStruct((E,), f32)),
        mesh=_sc_mesh(),
        scratch_types=[
            pltpu.VMEM((NNODE,), f32),
            pltpu.VMEM((_EPW + 16,), i32),
            pltpu.VMEM((_EPW + 16,), i32),
            pltpu.VMEM((_EPW + 16,), i32),
            pltpu.VMEM((_EPW + 16,), f32),
            pltpu.VMEM((_EPW + 16,), f32),
        ],
        compiler_params=_NLP,
    )
    return fn(y, e0, e1, negs)


def kernel(labels, autoenc_skip0, autoenc_skip1, edges_nn, negs,
           W0, b0, W1, b1, w_lin):
    off = jnp.array([0, NSP, 0, NSP], i32).reshape(B, 1)
    labels2 = (labels.reshape(B, NPIX).astype(i32) + off).reshape(B * NPIX)
    e0 = edges_nn[0].astype(i32)
    e1 = edges_nn[1].astype(i32)
    negs = negs.astype(i32)

    u = _upsample_to_rows(autoenc_skip0, autoenc_skip1)
    u4 = u.reshape(8, NPIX, 2, HALF)

    zacc = jnp.zeros((ACCROWS, HALF), f32)
    zmp = jnp.zeros((MPROWS, HALF), f32)

    ssum, asum = _sc_pool(u4, labels2, _aux_rows(), zacc)
    xw0, coords = _compute_xw0(ssum, asum, W0)
    ea = _sc_edge_attr(e0, e1, coords.reshape(NNODE * 8))

    agg0 = _sc_message_pass(xw0.reshape(2 * NNODE, HALF), e0, e1, ea, zmp)
    xw1 = _compute_xw1(agg0, ssum, asum, b0, W1)
    agg1 = _sc_message_pass(xw1.reshape(2 * NNODE, HALF), e0, e1, ea, zmp)
    y = _compute_y(agg1, b1, w_lin)

    dan, dap = _sc_scores(y.reshape(NNODE), e0, e1, negs)
    return (dan.reshape(E, 1), dap.reshape(E, 1), ea)


# speedup vs baseline: 3.8697x; 1.1533x over previous
import functools

import jax
import jax.numpy as jnp
from jax import lax
from jax.experimental import pallas as pl
from jax.experimental.pallas import tpu as pltpu
from jax.experimental.pallas import tpu_sc as plsc

SIGMA = 0.05
MIX = 0.5
NSP = 2500
B = 4
C = 256
H = 56
OH = 112
NPIX = OH * OH
NNODE = B * NSP
E = 160000
ACCROWS = 5120
MPROWS = 10240
HALF = 128

f32 = jnp.float32
i32 = jnp.int32

_NLP = pltpu.CompilerParams(needs_layout_passes=False)


def _lerp_matrix():
    ys = jnp.linspace(0.0, H - 1.0, OH)
    y0 = jnp.floor(ys).astype(i32)
    y1 = jnp.minimum(y0 + 1, H - 1)
    wy = ys - y0.astype(f32)
    r = jnp.zeros((OH, H), f32)
    rows = jnp.arange(OH)
    r = r.at[rows, y0].add(1.0 - wy)
    r = r.at[rows, y1].add(wy)
    return r


def _aux_rows():
    ii = jnp.repeat(jnp.arange(OH, dtype=f32), OH) / (OH - 1.0)
    jj = jnp.tile(jnp.arange(OH, dtype=f32), OH) / (OH - 1.0)
    a = jnp.zeros((NPIX, HALF), f32)
    return a.at[:, 0].set(1.0).at[:, 1].set(ii).at[:, 2].set(jj)


def _mm_kernel(a_ref, b_ref, o_ref):
    o_ref[...] = jnp.dot(a_ref[...], b_ref[...],
                         preferred_element_type=f32,
                         precision=lax.Precision.DEFAULT)


def _xlerp(x2):
    m = x2.shape[0]
    blk = m // 32
    return pl.pallas_call(
        _mm_kernel,
        grid=(32,),
        in_specs=[pl.BlockSpec((blk, H), lambda i: (i, 0)),
                  pl.BlockSpec((H, OH), lambda i: (0, 0))],
        out_specs=pl.BlockSpec((blk, OH), lambda i: (i, 0)),
        out_shape=jax.ShapeDtypeStruct((m, OH), f32),
    )(x2, _lerp_matrix().T)


def _transpose_kernel(a_ref, e_ref, o_ref):
    o_ref[0] = lax.dot_general(a_ref[0], e_ref[...],
                               (((0,), (0,)), ((), ())),
                               preferred_element_type=f32,
                               precision=lax.Precision.DEFAULT)


def _transpose8(a):
    k = a.shape[2]
    kb = k // 7
    return pl.pallas_call(
        _transpose_kernel,
        grid=(8, 7),
        in_specs=[pl.BlockSpec((1, C, kb), lambda i, j: (i, 0, j)),
                  pl.BlockSpec((C, C), lambda i, j: (0, 0))],
        out_specs=pl.BlockSpec((1, kb, C), lambda i, j: (i, j, 0)),
        out_shape=jax.ShapeDtypeStruct((8, k, C), f32),
    )(a, jnp.eye(C, dtype=f32))


def _ylerp_kernel(r_ref, a_ref, o_ref):
    o_ref[0] = jnp.dot(r_ref[...], a_ref[0],
                       preferred_element_type=f32,
                       precision=lax.Precision.DEFAULT)


def _ylerp(a):
    k = a.shape[2]
    kb = k // 4
    return pl.pallas_call(
        _ylerp_kernel,
        grid=(8, 4),
        in_specs=[pl.BlockSpec((OH, H), lambda i, j: (0, 0)),
                  pl.BlockSpec((1, H, kb), lambda i, j: (i, 0, j))],
        out_specs=pl.BlockSpec((1, OH, kb), lambda i, j: (i, 0, j)),
        out_shape=jax.ShapeDtypeStruct((8, OH, k), f32),
    )(_lerp_matrix(), a)


def _upsample_to_rows(skip0, skip1):
    x = jnp.concatenate([skip0.reshape(B, C, H * H),
                         skip1.reshape(B, C, H * H)], axis=0)
    x2 = x.reshape(8 * C * H, H)
    a = _xlerp(x2)
    a = a.reshape(8, C, H * OH)
    at = _transpose8(a)
    at = at.reshape(8, H, OH * C)
    u = _ylerp(at)
    return u.reshape(8, NPIX, C)


def _xw0_kernel(s_ref, a_ref, w_ref, o_ref, c_ref):
    icnt = 1.0 / jnp.maximum(a_ref[:, 0:1], 1.0)
    xw = (jnp.dot(s_ref[0, 0], w_ref[:HALF], preferred_element_type=f32,
                  precision=lax.Precision.DEFAULT)
          + jnp.dot(s_ref[0, 1], w_ref[HALF:], preferred_element_type=f32,
                    precision=lax.Precision.DEFAULT)) * icnt
    o_ref[0] = xw[:, :HALF]
    o_ref[1] = xw[:, HALF:]
    z = jnp.zeros((a_ref.shape[0], 6), f32)
    c_ref[...] = jnp.concatenate(
        [a_ref[:, 1:2] * icnt, a_ref[:, 2:3] * icnt, z], axis=1)


def _xw1_kernel(g_ref, s_ref, a_ref, b_ref, w_ref, o_ref):
    icnt = 1.0 / jnp.maximum(a_ref[:, 0:1], 1.0)
    g = jnp.concatenate([g_ref[0], g_ref[1]], axis=1)
    s = jnp.concatenate([s_ref[0, 0], s_ref[0, 1]], axis=1)
    x2 = ((1.0 - MIX) * jnp.maximum(g + b_ref[...], 0.0)
          + MIX * icnt * s)
    xw = jnp.dot(x2, w_ref[...], preferred_element_type=f32,
                 precision=lax.Precision.DEFAULT)
    o_ref[0] = xw[:, :HALF]
    o_ref[1] = xw[:, HALF:]


def _y_kernel(g_ref, b_ref, wl_ref, o_ref):
    g = jnp.concatenate([g_ref[0], g_ref[1]], axis=1)
    x3 = jnp.maximum(g + b_ref[...], 0.0)
    o_ref[...] = jnp.sum(x3 * wl_ref[...], axis=1, keepdims=True)


_RB = 2000


def _compute_xw0(ssum, asum, w0):
    return pl.pallas_call(
        _xw0_kernel,
        grid=(NNODE // _RB,),
        in_specs=[pl.BlockSpec((1, 2, _RB, HALF), lambda i: (0, 0, i, 0)),
                  pl.BlockSpec((_RB, HALF), lambda i: (i, 0)),
                  pl.BlockSpec((C, C), lambda i: (0, 0))],
        out_specs=(pl.BlockSpec((2, _RB, HALF), lambda i: (0, i, 0)),
                   pl.BlockSpec((_RB, 8), lambda i: (i, 0))),
        out_shape=(jax.ShapeDtypeStruct((2, NNODE, HALF), f32),
                   jax.ShapeDtypeStruct((NNODE, 8), f32)),
    )(ssum, asum, w0)


def _compute_xw1(agg0, ssum, asum, b0, w1):
    return pl.pallas_call(
        _xw1_kernel,
        grid=(NNODE // _RB,),
        in_specs=[pl.BlockSpec((2, _RB, HALF), lambda i: (0, i, 0)),
                  pl.BlockSpec((1, 2, _RB, HALF), lambda i: (1, 0, i, 0)),
                  pl.BlockSpec((_RB, HALF), lambda i: (i, 0)),
                  pl.BlockSpec((1, C), lambda i: (0, 0)),
                  pl.BlockSpec((C, C), lambda i: (0, 0))],
        out_specs=pl.BlockSpec((2, _RB, HALF), lambda i: (0, i, 0)),
        out_shape=jax.ShapeDtypeStruct((2, NNODE, HALF), f32),
    )(agg0, ssum, asum, b0.reshape(1, C), w1)


def _compute_y(agg1, b1, w_lin):
    return pl.pallas_call(
        _y_kernel,
        grid=(NNODE // _RB,),
        in_specs=[pl.BlockSpec((2, _RB, HALF), lambda i: (0, i, 0)),
                  pl.BlockSpec((1, C), lambda i: (0, 0)),
                  pl.BlockSpec((1, C), lambda i: (0, 0))],
        out_specs=pl.BlockSpec((_RB, 1), lambda i: (i, 0)),
        out_shape=jax.ShapeDtypeStruct((NNODE, 1), f32),
    )(agg1, b1.reshape(1, C), w_lin.reshape(1, C))


@functools.cache
def _sc_mesh():
    return plsc.VectorSubcoreMesh(core_axis_name="c", subcore_axis_name="s")


_PCH = 112
_PPT = NPIX // 16
_ZR = ACCROWS // 16


def _pool_body(u_hbm, lab_hbm, aux_hbm, zacc_hbm, ssum_hbm, asum_hbm,
               acc, lab0, row0, gs0, ss0, lab1, row1, gs1, ss1):
    cid = lax.axis_index("c")
    sid = lax.axis_index("s")
    bufs = ((lab0, row0, gs0, ss0), (lab1, row1, gs1, ss1))
    gdesc = [None, None]
    sdesc = [None, None]

    def fetch(chunk, b):
        img, src_at, base = chunk
        labv, rows, gs, _ = bufs[b]
        if sdesc[b] is not None:
            sdesc[b].wait()
            sdesc[b] = None
        pltpu.sync_copy(lab_hbm.at[pl.ds(img * NPIX + base, _PCH)], labv)
        gdesc[b] = pltpu.async_copy(src_at(base), rows, gs)

    def scatter(b):
        labv, rows, _, ss = bufs[b]
        gdesc[b].wait()
        gdesc[b] = None
        sdesc[b] = pltpu.async_copy(rows, acc.at[labv], ss, add=True)

    def run_phase(chunks):
        fetch(chunks[0], 0)
        for i in range(len(chunks)):
            if i + 1 < len(chunks):
                fetch(chunks[i + 1], (i + 1) % 2)
            scatter(i % 2)
        for b in range(2):
            if sdesc[b] is not None:
                sdesc[b].wait()
                sdesc[b] = None

    def drain(out_at):
        pltpu.sync_copy(acc.at[pl.ds(sid * 312, 312)],
                        out_at(sid * 312, 312))

        @pl.when(sid == 0)
        def _():
            pltpu.sync_copy(acc.at[pl.ds(4992, 8)], out_at(4992, 8))

    def phase_chunks(src_for_img):
        chunks = []
        for bl in range(2):
            img = cid * 2 + bl
            src_at = src_for_img(bl)
            for k in range(_PPT // _PCH):
                chunks.append((img, src_at, sid * _PPT + k * _PCH))
        return chunks

    for m in range(2):
        for h in range(2):
            pltpu.sync_copy(zacc_hbm.at[pl.ds(sid * _ZR, _ZR)],
                            acc.at[pl.ds(sid * _ZR, _ZR)])
            plsc.subcore_barrier()

            def usrc(bl, m=m, h=h):
                mi = m * 4 + cid * 2 + bl
                return lambda base: u_hbm.at[mi, pl.ds(base, _PCH), h]

            run_phase(phase_chunks(usrc))
            plsc.subcore_barrier()
            drain(lambda r, n: ssum_hbm.at[m, h, pl.ds(cid * 5000 + r, n)])
            plsc.subcore_barrier()

    pltpu.sync_copy(zacc_hbm.at[pl.ds(sid * _ZR, _ZR)],
                    acc.at[pl.ds(sid * _ZR, _ZR)])
    plsc.subcore_barrier()
    run_phase(phase_chunks(
        lambda bl: lambda base: aux_hbm.at[pl.ds(base, _PCH)]))
    plsc.subcore_barrier()
    drain(lambda r, n: asum_hbm.at[pl.ds(cid * 5000 + r, n)])


def _sc_pool(u4, labels2, aux, zacc):
    fn = pl.kernel(
        _pool_body,
        out_type=(jax.ShapeDtypeStruct((2, 2, NNODE, HALF), f32),
                  jax.ShapeDtypeStruct((NNODE, HALF), f32)),
        mesh=_sc_mesh(),
        scratch_types=[
            pltpu.VMEM_SHARED((ACCROWS, HALF), f32),
            pltpu.VMEM((_PCH,), i32),
            pltpu.VMEM((_PCH, HALF), f32),
            pltpu.SemaphoreType.DMA,
            pltpu.SemaphoreType.DMA,
            pltpu.VMEM((_PCH,), i32),
            pltpu.VMEM((_PCH, HALF), f32),
            pltpu.SemaphoreType.DMA,
            pltpu.SemaphoreType.DMA,
        ],
        compiler_params=_NLP,
    )
    return fn(u4, labels2, aux, zacc)


_EPW = E // 32
_EVR = _EPW // 16


def _sanitize_tail(ref):
    lanes = lax.iota(i32, 16)
    v = ref[pl.ds(4992, 16)]
    ref[pl.ds(4992, 16)] = jnp.where(lanes < 8, v, 0)


def _eattr_body(e0_hbm, e1_hbm, tab_hbm, ea_hbm, tab, e0v, e1v, outv):
    cid = lax.axis_index("c")
    sid = lax.axis_index("s")
    wid = sid * 2 + cid
    base = wid * _EPW
    pltpu.sync_copy(tab_hbm, tab)
    pltpu.sync_copy(e0_hbm.at[pl.ds(base, _EPW)], e0v.at[pl.ds(0, _EPW)])
    pltpu.sync_copy(e1_hbm.at[pl.ds(base, _EPW)], e1v.at[pl.ds(0, _EPW)])
    _sanitize_tail(e0v)
    _sanitize_tail(e1v)

    def body(i, carry):
        sl = pl.ds(i * 16, 16)
        a = e0v[sl] * 8
        b = e1v[sl] * 8
        dx = plsc.load_gather(tab, [a]) - plsc.load_gather(tab, [b])
        dy = plsc.load_gather(tab, [a + 1]) - plsc.load_gather(tab, [b + 1])
        outv[sl] = jnp.exp(-(dx * dx + dy * dy) * (1.0 / SIGMA))
        return carry

    lax.fori_loop(0, _EVR + 1, body, 0)
    pltpu.sync_copy(outv.at[pl.ds(0, _EPW)], ea_hbm.at[pl.ds(base, _EPW)])


def _sc_edge_attr(e0, e1, tabflat):
    fn = pl.kernel(
        _eattr_body,
        out_type=jax.ShapeDtypeStruct((E,), f32),
        mesh=_sc_mesh(),
        scratch_types=[
            pltpu.VMEM((NNODE * 8,), f32),
            pltpu.VMEM((_EPW + 16,), i32),
            pltpu.VMEM((_EPW + 16,), i32),
            pltpu.VMEM((_EPW + 16,), f32),
        ],
        compiler_params=_NLP,
    )
    return fn(e0, e1, tabflat)


_ECH = 80
_EPT = E // 16
_NCH = _EPT // _ECH


def _mp_body(xw_hbm, e0_hbm, e1_hbm, ea_hbm, zmp_hbm, agg_hbm, acc,
             s0, d0, w0v, r0, gs0, ss0, as0,
             s1, d1, w1v, r1, gs1, ss1, as1,
             s2, d2, w2v, r2, gs2, ss2, as2):
    cid = lax.axis_index("c")
    sid = lax.axis_index("s")
    pltpu.sync_copy(zmp_hbm.at[pl.ds(sid * 640, 640)],
                    acc.at[pl.ds(sid * 640, 640)])
    plsc.subcore_barrier()
    bufs = ((s0, d0, w0v, r0, gs0, ss0, as0),
            (s1, d1, w1v, r1, gs1, ss1, as1),
            (s2, d2, w2v, r2, gs2, ss2, as2))

    def fetch(k, b):
        sv, dv, wv, rw, gs, _, asem = bufs[b]
        base = sid * _EPT + k * _ECH
        pltpu.async_copy(e1_hbm.at[pl.ds(base, _ECH)], dv, asem)
        pltpu.async_copy(ea_hbm.at[pl.ds(base, _ECH)], wv, asem)
        pltpu.sync_copy(e0_hbm.at[pl.ds(base, _ECH)], sv)
        for t in range(_ECH // 16):
            sv[pl.ds(t * 16, 16)] = sv[pl.ds(t * 16, 16)] + cid * NNODE
        pltpu.async_copy(xw_hbm.at[sv], rw, gs)

    def compute(b):
        sv, dv, wv, rw, gs, ss, asem = bufs[b]
        base = pl.ds(0, _ECH)
        pltpu.make_async_copy(e1_hbm.at[base], dv, asem).wait()
        pltpu.make_async_copy(ea_hbm.at[base], wv, asem).wait()
        pltpu.make_async_copy(xw_hbm.at[sv], rw, gs).wait()

        def edge(j, c2):
            s = plsc.load_gather(wv, [jnp.full((16,), 0, i32) + j])
            for q in range(HALF // 16):
                sl = pl.ds(q * 16, 16)
                rw[j, sl] = rw[j, sl] * s
            return c2

        lax.fori_loop(0, _ECH, edge, 0)
        pltpu.async_copy(rw, acc.at[dv], ss, add=True)

    def wait_scatter(b):
        _, dv, _, rw, _, ss, _ = bufs[b]
        pltpu.make_async_copy(rw, acc.at[dv], ss).wait()

    def step(c, b, f, wait_sc):
        if f is not None:
            if wait_sc:
                wait_scatter(f % 3)
            fetch(f, f % 3)
        compute(b)

    fetch(0, 0)
    fetch(1, 1)
    step(0, 0, 2, False)
    step(1, 1, 3, True)
    step(2, 2, 4, True)

    def k3body(k3, carry):
        c = 3 * k3
        for j in range(3):
            wait_scatter((j + 2) % 3)
            fetch(c + j + 2, (j + 2) % 3)
            compute(j)
        return carry

    lax.fori_loop(1, (_NCH - 5) // 3 + 1, k3body, 0)
    compute(0)
    compute(1)
    wait_scatter(0)
    wait_scatter(1)
    wait_scatter(2)
    plsc.subcore_barrier()
    pltpu.sync_copy(acc.at[pl.ds(sid * 624, 624)],
                    agg_hbm.at[cid, pl.ds(sid * 624, 624)])

    @pl.when(sid == 0)
    def _():
        pltpu.sync_copy(acc.at[pl.ds(9984, 16)],
                        agg_hbm.at[cid, pl.ds(9984, 16)])


def _sc_message_pass(xwflat, e0, e1, ea, zmp):
    fn = pl.kernel(
        _mp_body,
        out_type=jax.ShapeDtypeStruct((2, NNODE, HALF), f32),
        mesh=_sc_mesh(),
        scratch_types=[
            pltpu.VMEM_SHARED((MPROWS, HALF), f32),
            pltpu.VMEM((_ECH,), i32),
            pltpu.VMEM((_ECH,), i32),
            pltpu.VMEM((_ECH,), f32),
            pltpu.VMEM((_ECH, HALF), f32),
            pltpu.SemaphoreType.DMA,
            pltpu.SemaphoreType.DMA,
            pltpu.SemaphoreType.DMA,
            pltpu.VMEM((_ECH,), i32),
            pltpu.VMEM((_ECH,), i32),
            pltpu.VMEM((_ECH,), f32),
            pltpu.VMEM((_ECH, HALF), f32),
            pltpu.SemaphoreType.DMA,
            pltpu.SemaphoreType.DMA,
            pltpu.SemaphoreType.DMA,
            pltpu.VMEM((_ECH,), i32),
            pltpu.VMEM((_ECH,), i32),
            pltpu.VMEM((_ECH,), f32),
            pltpu.VMEM((_ECH, HALF), f32),
            pltpu.SemaphoreType.DMA,
            pltpu.SemaphoreType.DMA,
            pltpu.SemaphoreType.DMA,
        ],
        compiler_params=_NLP,
    )
    return fn(xwflat, e0, e1, ea, zmp)


def _scores_body(y_hbm, e0_hbm, e1_hbm, negs_hbm, dan_hbm, dap_hbm,
                 yv, e0v, e1v, env, danv, dapv):
    cid = lax.axis_index("c")
    sid = lax.axis_index("s")
    wid = sid * 2 + cid
    base = wid * _EPW
    pltpu.sync_copy(y_hbm, yv)
    pltpu.sync_copy(e0_hbm.at[pl.ds(base, _EPW)], e0v.at[pl.ds(0, _EPW)])
    pltpu.sync_copy(e1_hbm.at[pl.ds(base, _EPW)], e1v.at[pl.ds(0, _EPW)])
    pltpu.sync_copy(negs_hbm.at[pl.ds(base, _EPW)], env.at[pl.ds(0, _EPW)])
    _sanitize_tail(e0v)
    _sanitize_tail(e1v)
    _sanitize_tail(env)

    def body(i, carry):
        sl = pl.ds(i * 16, 16)
        s0 = plsc.load_gather(yv, [e0v[sl]])
        s1 = plsc.load_gather(yv, [e1v[sl]])
        sn = plsc.load_gather(yv, [env[sl]])
        dapv[sl] = 1.0 / (1.0 + jnp.exp(s1 - s0))
        danv[sl] = 1.0 / (1.0 + jnp.exp(sn - s0))
        return carry

    lax.fori_loop(0, _EVR + 1, body, 0)
    pltpu.sync_copy(danv.at[pl.ds(0, _EPW)], dan_hbm.at[pl.ds(base, _EPW)])
    pltpu.sync_copy(dapv.at[pl.ds(0, _EPW)], dap_hbm.at[pl.ds(base, _EPW)])


def _sc_scores(y, e0, e1, negs):
    fn = pl.kernel(
        _scores_body,
        out_type=(jax.ShapeDtypeStruct((E,), f32),
                  jax.ShapeDtypeStruct((E,), f32)),
        mesh=_sc_mesh(),
        scratch_types=[
            pltpu.VMEM((NNODE,), f32),
            pltpu.VMEM((_EPW + 16,), i32),
            pltpu.VMEM((_EPW + 16,), i32),
            pltpu.VMEM((_EPW + 16,), i32),
            pltpu.VMEM((_EPW + 16,), f32),
            pltpu.VMEM((_EPW + 16,), f32),
        ],
        compiler_params=_NLP,
    )
    return fn(y, e0, e1, negs)


def kernel(labels, autoenc_skip0, autoenc_skip1, edges_nn, negs,
           W0, b0, W1, b1, w_lin):
    off = jnp.array([0, NSP, 0, NSP], i32).reshape(B, 1)
    labels2 = (labels.reshape(B, NPIX).astype(i32) + off).reshape(B * NPIX)
    e0 = edges_nn[0].astype(i32)
    e1 = edges_nn[1].astype(i32)
    negs = negs.astype(i32)

    u = _upsample_to_rows(autoenc_skip0, autoenc_skip1)
    u4 = u.reshape(8, NPIX, 2, HALF)

    zacc = jnp.zeros((ACCROWS, HALF), f32)
    zmp = jnp.zeros((MPROWS, HALF), f32)

    ssum, asum = _sc_pool(u4, labels2, _aux_rows(), zacc)
    xw0, coords = _compute_xw0(ssum, asum, W0)
    ea = _sc_edge_attr(e0, e1, coords.reshape(NNODE * 8))

    agg0 = _sc_message_pass(xw0.reshape(2 * NNODE, HALF), e0, e1, ea, zmp)
    xw1 = _compute_xw1(agg0, ssum, asum, b0, W1)
    agg1 = _sc_message_pass(xw1.reshape(2 * NNODE, HALF), e0, e1, ea, zmp)
    y = _compute_y(agg1, b1, w_lin)

    dan, dap = _sc_scores(y.reshape(NNODE), e0, e1, negs)
    return (dan.reshape(E, 1), dap.reshape(E, 1), ea)


# speedup vs baseline: 3.9020x; 1.0083x over previous
import functools

import jax
import jax.numpy as jnp
from jax import lax
from jax.experimental import pallas as pl
from jax.experimental.pallas import tpu as pltpu
from jax.experimental.pallas import tpu_sc as plsc

SIGMA = 0.05
MIX = 0.5
NSP = 2500
B = 4
C = 256
H = 56
OH = 112
NPIX = OH * OH
NNODE = B * NSP
E = 160000
ACCROWS = 5120
MPROWS = 10240
HALF = 128

f32 = jnp.float32
i32 = jnp.int32

_NLP = pltpu.CompilerParams(needs_layout_passes=False)


def _lerp_matrix():
    ys = jnp.linspace(0.0, H - 1.0, OH)
    y0 = jnp.floor(ys).astype(i32)
    y1 = jnp.minimum(y0 + 1, H - 1)
    wy = ys - y0.astype(f32)
    r = jnp.zeros((OH, H), f32)
    rows = jnp.arange(OH)
    r = r.at[rows, y0].add(1.0 - wy)
    r = r.at[rows, y1].add(wy)
    return r


def _aux_rows():
    ii = jnp.repeat(jnp.arange(OH, dtype=f32), OH) / (OH - 1.0)
    jj = jnp.tile(jnp.arange(OH, dtype=f32), OH) / (OH - 1.0)
    a = jnp.zeros((NPIX, HALF), f32)
    return a.at[:, 0].set(1.0).at[:, 1].set(ii).at[:, 2].set(jj)


def _mm_kernel(a_ref, b_ref, o_ref):
    o_ref[...] = jnp.dot(a_ref[...], b_ref[...],
                         preferred_element_type=f32,
                         precision=lax.Precision.DEFAULT)


def _xlerp(x2):
    m = x2.shape[0]
    blk = m // 32
    return pl.pallas_call(
        _mm_kernel,
        grid=(32,),
        in_specs=[pl.BlockSpec((blk, H), lambda i: (i, 0)),
                  pl.BlockSpec((H, OH), lambda i: (0, 0))],
        out_specs=pl.BlockSpec((blk, OH), lambda i: (i, 0)),
        out_shape=jax.ShapeDtypeStruct((m, OH), f32),
    )(x2, _lerp_matrix().T)


def _transpose_kernel(a_ref, e_ref, o_ref):
    o_ref[0] = lax.dot_general(a_ref[0], e_ref[...],
                               (((0,), (0,)), ((), ())),
                               preferred_element_type=f32,
                               precision=lax.Precision.DEFAULT)


def _transpose8(a):
    k = a.shape[2]
    kb = k // 7
    return pl.pallas_call(
        _transpose_kernel,
        grid=(8, 7),
        in_specs=[pl.BlockSpec((1, C, kb), lambda i, j: (i, 0, j)),
                  pl.BlockSpec((C, C), lambda i, j: (0, 0))],
        out_specs=pl.BlockSpec((1, kb, C), lambda i, j: (i, j, 0)),
        out_shape=jax.ShapeDtypeStruct((8, k, C), f32),
    )(a, jnp.eye(C, dtype=f32))


def _ylerp_kernel(r_ref, a_ref, o_ref):
    o_ref[0] = jnp.dot(r_ref[...], a_ref[0],
                       preferred_element_type=f32,
                       precision=lax.Precision.DEFAULT)


def _ylerp(a):
    k = a.shape[2]
    kb = k // 4
    return pl.pallas_call(
        _ylerp_kernel,
        grid=(8, 4),
        in_specs=[pl.BlockSpec((OH, H), lambda i, j: (0, 0)),
                  pl.BlockSpec((1, H, kb), lambda i, j: (i, 0, j))],
        out_specs=pl.BlockSpec((1, OH, kb), lambda i, j: (i, 0, j)),
        out_shape=jax.ShapeDtypeStruct((8, OH, k), f32),
    )(_lerp_matrix(), a)


def _upsample_to_rows(skip0, skip1):
    x = jnp.concatenate([skip0.reshape(B, C, H * H),
                         skip1.reshape(B, C, H * H)], axis=0)
    x2 = x.reshape(8 * C * H, H)
    a = _xlerp(x2)
    a = a.reshape(8, C, H * OH)
    at = _transpose8(a)
    at = at.reshape(8, H, OH * C)
    u = _ylerp(at)
    return u.reshape(8, NPIX, C)


def _xw0_kernel(s_ref, a_ref, w_ref, o_ref, c_ref):
    icnt = 1.0 / jnp.maximum(a_ref[:, 0:1], 1.0)
    xw = (jnp.dot(s_ref[0, 0], w_ref[:HALF], preferred_element_type=f32,
                  precision=lax.Precision.DEFAULT)
          + jnp.dot(s_ref[0, 1], w_ref[HALF:], preferred_element_type=f32,
                    precision=lax.Precision.DEFAULT)) * icnt
    o_ref[0] = xw[:, :HALF]
    o_ref[1] = xw[:, HALF:]
    z = jnp.zeros((a_ref.shape[0], 6), f32)
    c_ref[...] = jnp.concatenate(
        [a_ref[:, 1:2] * icnt, a_ref[:, 2:3] * icnt, z], axis=1)


def _xw1_kernel(g_ref, s_ref, a_ref, b_ref, w_ref, o_ref):
    icnt = 1.0 / jnp.maximum(a_ref[:, 0:1], 1.0)
    g = jnp.concatenate([g_ref[0], g_ref[1]], axis=1)
    s = jnp.concatenate([s_ref[0, 0], s_ref[0, 1]], axis=1)
    x2 = ((1.0 - MIX) * jnp.maximum(g + b_ref[...], 0.0)
          + MIX * icnt * s)
    xw = jnp.dot(x2, w_ref[...], preferred_element_type=f32,
                 precision=lax.Precision.DEFAULT)
    o_ref[0] = xw[:, :HALF]
    o_ref[1] = xw[:, HALF:]


def _y_kernel(g_ref, b_ref, wl_ref, o_ref):
    g = jnp.concatenate([g_ref[0], g_ref[1]], axis=1)
    x3 = jnp.maximum(g + b_ref[...], 0.0)
    o_ref[...] = jnp.sum(x3 * wl_ref[...], axis=1, keepdims=True)


_RB = 2000


def _compute_xw0(ssum, asum, w0):
    return pl.pallas_call(
        _xw0_kernel,
        grid=(NNODE // _RB,),
        in_specs=[pl.BlockSpec((1, 2, _RB, HALF), lambda i: (0, 0, i, 0)),
                  pl.BlockSpec((_RB, HALF), lambda i: (i, 0)),
                  pl.BlockSpec((C, C), lambda i: (0, 0))],
        out_specs=(pl.BlockSpec((2, _RB, HALF), lambda i: (0, i, 0)),
                   pl.BlockSpec((_RB, 8), lambda i: (i, 0))),
        out_shape=(jax.ShapeDtypeStruct((2, NNODE, HALF), f32),
                   jax.ShapeDtypeStruct((NNODE, 8), f32)),
    )(ssum, asum, w0)


def _compute_xw1(agg0, ssum, asum, b0, w1):
    return pl.pallas_call(
        _xw1_kernel,
        grid=(NNODE // _RB,),
        in_specs=[pl.BlockSpec((2, _RB, HALF), lambda i: (0, i, 0)),
                  pl.BlockSpec((1, 2, _RB, HALF), lambda i: (1, 0, i, 0)),
                  pl.BlockSpec((_RB, HALF), lambda i: (i, 0)),
                  pl.BlockSpec((1, C), lambda i: (0, 0)),
                  pl.BlockSpec((C, C), lambda i: (0, 0))],
        out_specs=pl.BlockSpec((2, _RB, HALF), lambda i: (0, i, 0)),
        out_shape=jax.ShapeDtypeStruct((2, NNODE, HALF), f32),
    )(agg0, ssum, asum, b0.reshape(1, C), w1)


def _compute_y(agg1, b1, w_lin):
    return pl.pallas_call(
        _y_kernel,
        grid=(NNODE // _RB,),
        in_specs=[pl.BlockSpec((2, _RB, HALF), lambda i: (0, i, 0)),
                  pl.BlockSpec((1, C), lambda i: (0, 0)),
                  pl.BlockSpec((1, C), lambda i: (0, 0))],
        out_specs=pl.BlockSpec((_RB, 1), lambda i: (i, 0)),
        out_shape=jax.ShapeDtypeStruct((NNODE, 1), f32),
    )(agg1, b1.reshape(1, C), w_lin.reshape(1, C))


@functools.cache
def _sc_mesh():
    return plsc.VectorSubcoreMesh(core_axis_name="c", subcore_axis_name="s")


_PCH = 112
_PPT = NPIX // 16
_ZR = ACCROWS // 16


def _pool_body(u_hbm, lab_hbm, aux_hbm, zacc_hbm, ssum_hbm, asum_hbm,
               acc, lab0, row0, gs0, ss0, ls0, lab1, row1, gs1, ss1, ls1):
    cid = lax.axis_index("c")
    sid = lax.axis_index("s")
    bufs = ((lab0, row0, gs0, ss0, ls0), (lab1, row1, gs1, ss1, ls1))
    gdesc = [None, None]
    sdesc = [None, None]

    def fetch(chunk, b):
        img, src_at, base = chunk
        labv, rows, gs, _, lsem = bufs[b]
        if sdesc[b] is not None:
            sdesc[b].wait()
            sdesc[b] = None
        pltpu.async_copy(lab_hbm.at[pl.ds(img * NPIX + base, _PCH)],
                         labv, lsem)
        gdesc[b] = pltpu.async_copy(src_at(base), rows, gs)

    def scatter(b):
        labv, rows, _, ss, lsem = bufs[b]
        gdesc[b].wait()
        gdesc[b] = None
        pltpu.make_async_copy(lab_hbm.at[pl.ds(0, _PCH)], labv, lsem).wait()
        sdesc[b] = pltpu.async_copy(rows, acc.at[labv], ss, add=True)

    def run_phase(chunks):
        fetch(chunks[0], 0)
        for i in range(len(chunks)):
            if i + 1 < len(chunks):
                fetch(chunks[i + 1], (i + 1) % 2)
            scatter(i % 2)
        for b in range(2):
            if sdesc[b] is not None:
                sdesc[b].wait()
                sdesc[b] = None

    def drain(out_at):
        pltpu.sync_copy(acc.at[pl.ds(sid * 312, 312)],
                        out_at(sid * 312, 312))

        @pl.when(sid == 0)
        def _():
            pltpu.sync_copy(acc.at[pl.ds(4992, 8)], out_at(4992, 8))

    def phase_chunks(src_for_img):
        chunks = []
        for bl in range(2):
            img = cid * 2 + bl
            src_at = src_for_img(bl)
            for k in range(_PPT // _PCH):
                chunks.append((img, src_at, sid * _PPT + k * _PCH))
        return chunks

    for m in range(2):
        for h in range(2):
            pltpu.sync_copy(zacc_hbm.at[pl.ds(sid * _ZR, _ZR)],
                            acc.at[pl.ds(sid * _ZR, _ZR)])
            plsc.subcore_barrier()

            def usrc(bl, m=m, h=h):
                mi = m * 4 + cid * 2 + bl
                return lambda base: u_hbm.at[mi, pl.ds(base, _PCH), h]

            run_phase(phase_chunks(usrc))
            plsc.subcore_barrier()
            drain(lambda r, n: ssum_hbm.at[m, h, pl.ds(cid * 5000 + r, n)])
            plsc.subcore_barrier()

    pltpu.sync_copy(zacc_hbm.at[pl.ds(sid * _ZR, _ZR)],
                    acc.at[pl.ds(sid * _ZR, _ZR)])
    plsc.subcore_barrier()
    run_phase(phase_chunks(
        lambda bl: lambda base: aux_hbm.at[pl.ds(base, _PCH)]))
    plsc.subcore_barrier()
    drain(lambda r, n: asum_hbm.at[pl.ds(cid * 5000 + r, n)])


def _sc_pool(u4, labels2, aux, zacc):
    fn = pl.kernel(
        _pool_body,
        out_type=(jax.ShapeDtypeStruct((2, 2, NNODE, HALF), f32),
                  jax.ShapeDtypeStruct((NNODE, HALF), f32)),
        mesh=_sc_mesh(),
        scratch_types=[
            pltpu.VMEM_SHARED((ACCROWS, HALF), f32),
            pltpu.VMEM((_PCH,), i32),
            pltpu.VMEM((_PCH, HALF), f32),
            pltpu.SemaphoreType.DMA,
            pltpu.SemaphoreType.DMA,
            pltpu.SemaphoreType.DMA,
            pltpu.VMEM((_PCH,), i32),
            pltpu.VMEM((_PCH, HALF), f32),
            pltpu.SemaphoreType.DMA,
            pltpu.SemaphoreType.DMA,
            pltpu.SemaphoreType.DMA,
        ],
        compiler_params=_NLP,
    )
    return fn(u4, labels2, aux, zacc)


_EPW = E // 32
_EVR = _EPW // 16


def _sanitize_tail(ref):
    lanes = lax.iota(i32, 16)
    v = ref[pl.ds(4992, 16)]
    ref[pl.ds(4992, 16)] = jnp.where(lanes < 8, v, 0)


def _eattr_body(e0_hbm, e1_hbm, tab_hbm, ea_hbm, tab, e0v, e1v, outv):
    cid = lax.axis_index("c")
    sid = lax.axis_index("s")
    wid = sid * 2 + cid
    base = wid * _EPW
    pltpu.sync_copy(tab_hbm, tab)
    pltpu.sync_copy(e0_hbm.at[pl.ds(base, _EPW)], e0v.at[pl.ds(0, _EPW)])
    pltpu.sync_copy(e1_hbm.at[pl.ds(base, _EPW)], e1v.at[pl.ds(0, _EPW)])
    _sanitize_tail(e0v)
    _sanitize_tail(e1v)

    def body(i, carry):
        sl = pl.ds(i * 16, 16)
        a = e0v[sl] * 8
        b = e1v[sl] * 8
        dx = plsc.load_gather(tab, [a]) - plsc.load_gather(tab, [b])
        dy = plsc.load_gather(tab, [a + 1]) - plsc.load_gather(tab, [b + 1])
        outv[sl] = jnp.exp(-(dx * dx + dy * dy) * (1.0 / SIGMA))
        return carry

    lax.fori_loop(0, _EVR + 1, body, 0)
    pltpu.sync_copy(outv.at[pl.ds(0, _EPW)], ea_hbm.at[pl.ds(base, _EPW)])


def _sc_edge_attr(e0, e1, tabflat):
    fn = pl.kernel(
        _eattr_body,
        out_type=jax.ShapeDtypeStruct((E,), f32),
        mesh=_sc_mesh(),
        scratch_types=[
            pltpu.VMEM((NNODE * 8,), f32),
            pltpu.VMEM((_EPW + 16,), i32),
            pltpu.VMEM((_EPW + 16,), i32),
            pltpu.VMEM((_EPW + 16,), f32),
        ],
        compiler_params=_NLP,
    )
    return fn(e0, e1, tabflat)


_ECH = 80
_EPT = E // 16
_NCH = _EPT // _ECH


def _mp_body(xw_hbm, e0_hbm, e1_hbm, ea_hbm, zmp_hbm, agg_hbm, acc,
             s0, d0, w0v, r0, gs0, ss0, as0,
             s1, d1, w1v, r1, gs1, ss1, as1,
             s2, d2, w2v, r2, gs2, ss2, as2):
    cid = lax.axis_index("c")
    sid = lax.axis_index("s")
    pltpu.sync_copy(zmp_hbm.at[pl.ds(sid * 640, 640)],
                    acc.at[pl.ds(sid * 640, 640)])
    plsc.subcore_barrier()
    bufs = ((s0, d0, w0v, r0, gs0, ss0, as0),
            (s1, d1, w1v, r1, gs1, ss1, as1),
            (s2, d2, w2v, r2, gs2, ss2, as2))

    def fetch(k, b):
        sv, dv, wv, rw, gs, _, asem = bufs[b]
        base = sid * _EPT + k * _ECH
        pltpu.async_copy(e1_hbm.at[pl.ds(base, _ECH)], dv, asem)
        pltpu.async_copy(ea_hbm.at[pl.ds(base, _ECH)], wv, asem)
        pltpu.sync_copy(e0_hbm.at[pl.ds(base, _ECH)], sv)
        for t in range(_ECH // 16):
            sv[pl.ds(t * 16, 16)] = sv[pl.ds(t * 16, 16)] + cid * NNODE
        pltpu.async_copy(xw_hbm.at[sv], rw, gs)

    def compute(b):
        sv, dv, wv, rw, gs, ss, asem = bufs[b]
        base = pl.ds(0, _ECH)
        pltpu.make_async_copy(e1_hbm.at[base], dv, asem).wait()
        pltpu.make_async_copy(ea_hbm.at[base], wv, asem).wait()
        pltpu.make_async_copy(xw_hbm.at[sv], rw, gs).wait()

        def edge(j, c2):
            s = plsc.load_gather(wv, [jnp.full((16,), 0, i32) + j])
            for q in range(HALF // 16):
                sl = pl.ds(q * 16, 16)
                rw[j, sl] = rw[j, sl] * s
            return c2

        lax.fori_loop(0, _ECH, edge, 0)
        pltpu.async_copy(rw, acc.at[dv], ss, add=True)

    def wait_scatter(b):
        _, dv, _, rw, _, ss, _ = bufs[b]
        pltpu.make_async_copy(rw, acc.at[dv], ss).wait()

    def step(c, b, f, wait_sc):
        if f is not None:
            if wait_sc:
                wait_scatter(f % 3)
            fetch(f, f % 3)
        compute(b)

    fetch(0, 0)
    fetch(1, 1)
    step(0, 0, 2, False)
    step(1, 1, 3, True)
    step(2, 2, 4, True)

    def k3body(k3, carry):
        c = 3 * k3
        for j in range(3):
            wait_scatter((j + 2) % 3)
            fetch(c + j + 2, (j + 2) % 3)
            compute(j)
        return carry

    lax.fori_loop(1, (_NCH - 5) // 3 + 1, k3body, 0)
    compute(0)
    compute(1)
    wait_scatter(0)
    wait_scatter(1)
    wait_scatter(2)
    plsc.subcore_barrier()
    pltpu.sync_copy(acc.at[pl.ds(sid * 624, 624)],
                    agg_hbm.at[cid, pl.ds(sid * 624, 624)])

    @pl.when(sid == 0)
    def _():
        pltpu.sync_copy(acc.at[pl.ds(9984, 16)],
                        agg_hbm.at[cid, pl.ds(9984, 16)])


def _sc_message_pass(xwflat, e0, e1, ea, zmp):
    fn = pl.kernel(
        _mp_body,
        out_type=jax.ShapeDtypeStruct((2, NNODE, HALF), f32),
        mesh=_sc_mesh(),
        scratch_types=[
            pltpu.VMEM_SHARED((MPROWS, HALF), f32),
            pltpu.VMEM((_ECH,), i32),
            pltpu.VMEM((_ECH,), i32),
            pltpu.VMEM((_ECH,), f32),
            pltpu.VMEM((_ECH, HALF), f32),
            pltpu.SemaphoreType.DMA,
            pltpu.SemaphoreType.DMA,
            pltpu.SemaphoreType.DMA,
            pltpu.VMEM((_ECH,), i32),
            pltpu.VMEM((_ECH,), i32),
            pltpu.VMEM((_ECH,), f32),
            pltpu.VMEM((_ECH, HALF), f32),
            pltpu.SemaphoreType.DMA,
            pltpu.SemaphoreType.DMA,
            pltpu.SemaphoreType.DMA,
            pltpu.VMEM((_ECH,), i32),
            pltpu.VMEM((_ECH,), i32),
            pltpu.VMEM((_ECH,), f32),
            pltpu.VMEM((_ECH, HALF), f32),
            pltpu.SemaphoreType.DMA,
            pltpu.SemaphoreType.DMA,
            pltpu.SemaphoreType.DMA,
        ],
        compiler_params=_NLP,
    )
    return fn(xwflat, e0, e1, ea, zmp)


def _scores_body(y_hbm, e0_hbm, e1_hbm, negs_hbm, dan_hbm, dap_hbm,
                 yv, e0v, e1v, env, danv, dapv):
    cid = lax.axis_index("c")
    sid = lax.axis_index("s")
    wid = sid * 2 + cid
    base = wid * _EPW
    pltpu.sync_copy(y_hbm, yv)
    pltpu.sync_copy(e0_hbm.at[pl.ds(base, _EPW)], e0v.at[pl.ds(0, _EPW)])
    pltpu.sync_copy(e1_hbm.at[pl.ds(base, _EPW)], e1v.at[pl.ds(0, _EPW)])
    pltpu.sync_copy(negs_hbm.at[pl.ds(base, _EPW)], env.at[pl.ds(0, _EPW)])
    _sanitize_tail(e0v)
    _sanitize_tail(e1v)
    _sanitize_tail(env)

    def body(i, carry):
        sl = pl.ds(i * 16, 16)
        s0 = plsc.load_gather(yv, [e0v[sl]])
        s1 = plsc.load_gather(yv, [e1v[sl]])
        sn = plsc.load_gather(yv, [env[sl]])
        dapv[sl] = 1.0 / (1.0 + jnp.exp(s1 - s0))
        danv[sl] = 1.0 / (1.0 + jnp.exp(sn - s0))
        return carry

    lax.fori_loop(0, _EVR + 1, body, 0)
    pltpu.sync_copy(danv.at[pl.ds(0, _EPW)], dan_hbm.at[pl.ds(base, _EPW)])
    pltpu.sync_copy(dapv.at[pl.ds(0, _EPW)], dap_hbm.at[pl.ds(base, _EPW)])


def _sc_scores(y, e0, e1, negs):
    fn = pl.kernel(
        _scores_body,
        out_type=(jax.ShapeDtypeStruct((E,), f32),
                  jax.ShapeDtypeStruct((E,), f32)),
        mesh=_sc_mesh(),
        scratch_types=[
            pltpu.VMEM((NNODE,), f32),
            pltpu.VMEM((_EPW + 16,), i32),
            pltpu.VMEM((_EPW + 16,), i32),
            pltpu.VMEM((_EPW + 16,), i32),
            pltpu.VMEM((_EPW + 16,), f32),
            pltpu.VMEM((_EPW + 16,), f32),
        ],
        compiler_params=_NLP,
    )
    return fn(y, e0, e1, negs)


def kernel(labels, autoenc_skip0, autoenc_skip1, edges_nn, negs,
           W0, b0, W1, b1, w_lin):
    off = jnp.array([0, NSP, 0, NSP], i32).reshape(B, 1)
    labels2 = (labels.reshape(B, NPIX).astype(i32) + off).reshape(B * NPIX)
    e0 = edges_nn[0].astype(i32)
    e1 = edges_nn[1].astype(i32)
    negs = negs.astype(i32)

    u = _upsample_to_rows(autoenc_skip0, autoenc_skip1)
    u4 = u.reshape(8, NPIX, 2, HALF)

    zacc = jnp.zeros((ACCROWS, HALF), f32)
    zmp = jnp.zeros((MPROWS, HALF), f32)

    ssum, asum = _sc_pool(u4, labels2, _aux_rows(), zacc)
    xw0, coords = _compute_xw0(ssum, asum, W0)
    ea = _sc_edge_attr(e0, e1, coords.reshape(NNODE * 8))

    agg0 = _sc_message_pass(xw0.reshape(2 * NNODE, HALF), e0, e1, ea, zmp)
    xw1 = _compute_xw1(agg0, ssum, asum, b0, W1)
    agg1 = _sc_message_pass(xw1.reshape(2 * NNODE, HALF), e0, e1, ea, zmp)
    y = _compute_y(agg1, b1, w_lin)

    dan, dap = _sc_scores(y.reshape(NNODE), e0, e1, negs)
    return (dan.reshape(E, 1), dap.reshape(E, 1), ea)
